# baseline (speedup 1.0000x reference)
import functools

import jax
import jax.numpy as jnp
from jax import lax
from jax.experimental import pallas as pl
from jax.experimental.pallas import tpu as pltpu
from jax.experimental.pallas import tpu_sc as plsc

N_NODES = 16384
D_FEAT = 512
HIDDEN = 1024
N_CLS = 128
BATCH = 2048
K = 64
EPS = 100.0 * float(jnp.finfo(jnp.float32).eps)
FMAX = float(jnp.finfo(jnp.float32).max)

NC, NS, L = 2, 16, 16
NW = NC * NS
ROWS_PER_W = BATCH // NW

CAP = 2048
GROUPS_PER_BLK = 16
BUFSZ = CAP + GROUPS_PER_BLK * L + L
N_BLKS = N_NODES // (GROUPS_PER_BLK * L)

T1 = 1.0 - 128.0 / N_NODES
T1_BITS = 0x3F7E0000
CAPF = 256
FGROUPS = (CAPF + 2 * L) // L
FBUF = 544
SENT = N_NODES
RSTRIDE = N_NODES + 128

def _sc_mesh():
    return plsc.VectorSubcoreMesh(
        core_axis_name="c", subcore_axis_name="s", num_cores=NC,
        num_subcores=NS)


def _iota16():
    return lax.iota(jnp.int32, L)


def _splat_i32(x):
    return jnp.full((L,), x, dtype=jnp.int32)


def _select_topk(rowv, base, cand, ptr):
    iota = _iota16()
    ptr_s = jnp.max(ptr)
    ngroups = (ptr_s + (L - 1)) // L

    def count_ge(tvec, strict):
        def body(g, cnt):
            bg = g * L
            idxg = cand[pl.ds(bg, L)]
            valid = (bg + iota) < ptr
            vg = plsc.load_gather(rowv, [idxg + base], mask=valid)
            hit = (vg > tvec if strict else vg >= tvec) & valid
            return cnt + plsc.all_reduce_population_count(hit)
        return lax.fori_loop(0, ngroups, body, _splat_i32(0))

    def bis_body(_, carry):
        lo, hi = carry
        mid = lax.shift_right_arithmetic(lo + hi, 1)
        cnt = count_ge(plsc.bitcast(mid, jnp.float32), False)
        ge = cnt >= K
        return jnp.where(ge, mid, lo), jnp.where(ge, hi, mid)

    lo, _ = lax.fori_loop(
        0, 30, bis_body, (_splat_i32(0), _splat_i32(0x3F800000)))
    tstar = plsc.bitcast(lo, jnp.float32)

    n_gt = count_ge(tstar, True)
    need = K - n_gt

    def compact(g, carry):
        wp, eqc = carry
        bg = g * L
        idxg = cand[pl.ds(bg, L)]
        valid = (bg + iota) < ptr
        vg = plsc.load_gather(rowv, [idxg + base], mask=valid)
        gt = (vg > tstar) & valid
        eq = (vg == tstar) & valid
        eq_rank = eqc + plsc.cumsum(eq.astype(jnp.int32))
        keep = gt | (eq & (eq_rank <= need))
        pos = wp + plsc.cumsum(keep.astype(jnp.int32)) - 1
        plsc.store_scatter(cand, [pos], idxg, mask=keep)
        return (wp + plsc.all_reduce_population_count(keep),
                eqc + plsc.all_reduce_population_count(eq))

    lax.fori_loop(0, ngroups, compact, (_splat_i32(0), _splat_i32(0)))
    return _splat_i32(K), tstar


def _sc_topk_body(rows_pw, ppr_hbm, idx_out, vals_out, rs_out, rowv2, cand,
                  candi, candvv, actg, idx_st, vals_st, rs_st, sem0, sem1):
    wid = lax.axis_index("s") * NC + lax.axis_index("c")
    row0 = wid * rows_pw
    iota = _iota16()
    lane0 = iota == 0
    sems = (sem0, sem1)
    for b in range(2):
        rowv2[pl.ds(b * RSTRIDE + SENT, L)] = jnp.full((L,), -1.0, jnp.float32)
    pltpu.async_copy(ppr_hbm.at[row0], rowv2.at[pl.ds(0, N_NODES)], sem0)

    def row_body(r, base):
        for j in range(FGROUPS):
            candi[pl.ds(j * L, L)] = _splat_i32(SENT)
            actg[pl.ds(j * L, L)] = _splat_i32(SENT // L)

        t1 = jnp.full((L,), T1, jnp.float32)

        def fblk(b, carry):
            acc0, acc1, nact = carry
            cnts = jnp.zeros((L,), jnp.int32)
            for gg in range(GROUPS_PER_BLK):
                off = (b * GROUPS_PER_BLK + gg) * L
                v = rowv2[pl.ds(base + off, L)]
                if gg % 2 == 0:
                    acc0 = acc0 + v
                else:
                    acc1 = acc1 + v
                pc = plsc.all_reduce_population_count(v > t1)
                cnts = jnp.where(iota == gg, pc, cnts)
            act = cnts > 0
            posn = nact + plsc.cumsum(act.astype(jnp.int32)) - 1
            posn = jnp.minimum(posn, FGROUPS * L - 1)
            plsc.store_scatter(actg, [posn], iota + b * L, mask=act)
            nact = nact + plsc.all_reduce_population_count(act)
            return acc0, acc1, nact

        acc0, acc1, nact = lax.fori_loop(
            0, N_BLKS, fblk,
            (jnp.zeros((L,), jnp.float32), jnp.zeros((L,), jnp.float32),
             _splat_i32(0)))
        acc = acc0 + acc1
        nact_s = jnp.minimum(jnp.max(nact), FGROUPS * L)

        def cblk(t, ptr):
            av = actg[pl.ds(t * L, L)]
            for j in range(L):
                gb = jnp.clip(av[j], 0, SENT // L) * L
                v = rowv2[pl.ds(base + gb, L)]
                m = v > t1
                pos = ptr + plsc.cumsum(m.astype(jnp.int32)) - 1
                pos = jnp.minimum(pos, FBUF - 1)
                plsc.store_scatter(candi, [pos], iota + gb, mask=m)
                ptr = ptr + plsc.all_reduce_population_count(m)
            return ptr

        ptr = lax.fori_loop(0, (nact_s + L - 1) // L, cblk, _splat_i32(0))
        ptr_s = jnp.max(ptr)
        ok = (ptr_s >= K) & (ptr_s <= CAPF) & (nact_s <= CAPF)

        @pl.when(ok)
        def _fast_select():
            for j in range(FGROUPS):
                ij = candi[pl.ds(j * L, L)]
                candvv[pl.ds(j * L, L)] = plsc.load_gather(rowv2, [ij + base])

            def count_s(tvec, strict):
                cnt = jnp.zeros((L,), jnp.int32)
                for j in range(FGROUPS):
                    vg = candvv[pl.ds(j * L, L)]
                    hit = vg > tvec if strict else vg >= tvec
                    cnt = cnt + hit.astype(jnp.int32)
                return jnp.sum(cnt, axis=0)

            def fbis(_, lohi):
                lo, hi = lohi
                mid = lax.shift_right_arithmetic(lo + hi, 1)
                tmid = lax.bitcast_convert_type(mid, jnp.float32)
                ge = count_s(jnp.full((L,), tmid), False) >= K
                return (jnp.where(ge, mid, lo), jnp.where(ge, hi, mid))

            lo, _ = lax.fori_loop(0, 17, fbis,
                                  (jnp.int32(T1_BITS), jnp.int32(0x3F800000)))
            tstar = jnp.full((L,), lax.bitcast_convert_type(lo, jnp.float32))
            need = K - _splat_i32(count_s(tstar, True))
            ngroups = (ptr_s + (L - 1)) // L

            def comp(g, carry):
                wp, eqc = carry
                base = g * L
                idxg = candi[pl.ds(base, L)]
                vg = candvv[pl.ds(base, L)]
                gt = vg > tstar
                eq = vg == tstar
                eq_rank = eqc + plsc.cumsum(eq.astype(jnp.int32))
                keep = gt | (eq & (eq_rank <= need))
                pos = wp + plsc.cumsum(keep.astype(jnp.int32)) - 1
                plsc.store_scatter(cand, [pos], idxg, mask=keep)
                return (wp + plsc.all_reduce_population_count(keep),
                        eqc + plsc.all_reduce_population_count(eq))

            lax.fori_loop(0, ngroups, comp,
                          (_splat_i32(0), _splat_i32(0)))

        @pl.when(jnp.logical_not(ok))
        def _fallback():
            def blk_body(b, carry):
                fptr, thr = carry
                for gg in range(GROUPS_PER_BLK):
                    off = (b * GROUPS_PER_BLK + gg) * L
                    v = rowv2[pl.ds(base + off, L)]
                    mask = v > thr
                    pos = fptr + plsc.cumsum(mask.astype(jnp.int32)) - 1
                    plsc.store_scatter(cand, [pos], iota + off, mask=mask)
                    fptr = fptr + plsc.all_reduce_population_count(mask)

                def rebuild():
                    return _select_topk(rowv2, base, cand, fptr)

                fptr, thr = lax.cond(jnp.max(fptr) > CAP, rebuild,
                                     lambda: (fptr, thr))
                return fptr, thr

            fptr, _ = lax.fori_loop(
                0, N_BLKS, blk_body,
                (_splat_i32(0), jnp.full((L,), -1.0, jnp.float32)))
            _select_topk(rowv2, base, cand, fptr)

        for j in range(K // L):
            idxg = cand[pl.ds(j * L, L)]
            vg = plsc.load_gather(rowv2, [idxg + base])
            idx_st[pl.ds(r * K + j * L, L)] = idxg
            vals_st[pl.ds(r * K + j * L, L)] = vg

        rsum = jnp.sum(acc, axis=0)
        plsc.store_scatter(rs_st, [_splat_i32(r)],
                           jnp.full((L,), rsum, jnp.float32), mask=lane0)

    def pair(t, _):
        for b in range(2):
            r = t * 2 + b

            @pl.when(r + 1 < rows_pw)
            def _():
                pltpu.async_copy(
                    ppr_hbm.at[row0 + r + 1],
                    rowv2.at[pl.ds(((b + 1) % 2) * RSTRIDE, N_NODES)],
                    sems[(b + 1) % 2])

            pltpu.make_async_copy(ppr_hbm.at[row0 + r],
                                  rowv2.at[pl.ds(b * RSTRIDE, N_NODES)],
                                  sems[b]).wait()
            row_body(r, b * RSTRIDE)
        return 0

    lax.fori_loop(0, rows_pw // 2, pair, 0)
    pltpu.sync_copy(idx_st, idx_out.at[pl.ds(row0 * K, rows_pw * K)])
    pltpu.sync_copy(vals_st, vals_out.at[pl.ds(row0 * K, rows_pw * K)])
    pltpu.sync_copy(rs_st, rs_out.at[pl.ds(row0, rows_pw)])


CH = 128
CHUNKS_PER_W = (BATCH * K) // NW // CH


def _sc_gather_body(chunks_pw, logits_hbm, idx_hbm, out_hbm, idx_v, rows_v,
                    sem0, sem1):
    wid = lax.axis_index("s") * NC + lax.axis_index("c")
    base = wid * chunks_pw * CH
    pltpu.sync_copy(idx_hbm.at[pl.ds(wid * chunks_pw, chunks_pw), :],
                    idx_v)
    sems = (sem0, sem1)
    pltpu.async_copy(logits_hbm.at[idx_v.at[0]], rows_v.at[0], sem0)

    def outer(t, _):
        for b in range(2):
            j = t * 2 + b

            @pl.when(j + 1 < chunks_pw)
            def _():
                pltpu.async_copy(logits_hbm.at[idx_v.at[j + 1]],
                                 rows_v.at[(b + 1) % 2], sems[(b + 1) % 2])

            pltpu.make_async_copy(logits_hbm.at[idx_v.at[j]], rows_v.at[b],
                                  sems[b]).wait()
            pltpu.sync_copy(rows_v.at[b],
                            out_hbm.at[pl.ds(base + j * CH, CH), :])
        return 0

    lax.fori_loop(0, chunks_pw // 2, outer, 0)


@functools.cache
def _sc_kernels(batch_s):
    sc_params = pltpu.CompilerParams(needs_layout_passes=False)
    rows_pw = batch_s // NW
    chunks_pw = (batch_s * K) // NW // CH
    topk = pl.kernel(
        functools.partial(_sc_topk_body, rows_pw),
        compiler_params=sc_params,
        out_type=(jax.ShapeDtypeStruct((batch_s * K,), jnp.int32),
                  jax.ShapeDtypeStruct((batch_s * K,), jnp.float32),
                  jax.ShapeDtypeStruct((batch_s,), jnp.float32)),
        mesh=_sc_mesh(),
        scratch_types=[
            pltpu.VMEM((2 * RSTRIDE,), jnp.float32),
            pltpu.VMEM((BUFSZ,), jnp.int32),
            pltpu.VMEM((FBUF,), jnp.int32),
            pltpu.VMEM((FGROUPS * L,), jnp.float32),
            pltpu.VMEM((FGROUPS * L,), jnp.int32),
            pltpu.VMEM((rows_pw * K,), jnp.int32),
            pltpu.VMEM((rows_pw * K,), jnp.float32),
            pltpu.VMEM((rows_pw,), jnp.float32),
            pltpu.SemaphoreType.DMA,
            pltpu.SemaphoreType.DMA,
        ],
    )
    gather = pl.kernel(
        functools.partial(_sc_gather_body, chunks_pw),
        compiler_params=sc_params,
        out_type=jax.ShapeDtypeStruct((batch_s * K, N_CLS), jnp.float32),
        mesh=_sc_mesh(),
        scratch_types=[
            pltpu.VMEM((chunks_pw, CH), jnp.int32),
            pltpu.VMEM((2, CH, N_CLS), jnp.float32),
            pltpu.SemaphoreType.DMA,
            pltpu.SemaphoreType.DMA,
        ],
    )
    return topk, gather


MLP_BLK = 512


def _mlp_body(x_ref, w0_ref, w1_ref, w2_ref, out_ref):
    h = jnp.dot(x_ref[...], w0_ref[...], preferred_element_type=jnp.float32)
    h = jnp.dot(jnp.maximum(h, 0.0), w1_ref[...],
                preferred_element_type=jnp.float32)
    out_ref[...] = jnp.dot(jnp.maximum(h, 0.0), w2_ref[...],
                           preferred_element_type=jnp.float32)


def _mlp(X, W0, W1, W2):
    return pl.pallas_call(
        _mlp_body,
        grid=(N_NODES // MLP_BLK,),
        in_specs=[
            pl.BlockSpec((MLP_BLK, D_FEAT), lambda i: (i, 0)),
            pl.BlockSpec((D_FEAT, HIDDEN), lambda i: (0, 0)),
            pl.BlockSpec((HIDDEN, HIDDEN), lambda i: (0, 0)),
            pl.BlockSpec((HIDDEN, N_CLS), lambda i: (0, 0)),
        ],
        out_specs=pl.BlockSpec((MLP_BLK, N_CLS), lambda i: (i, 0)),
        out_shape=jax.ShapeDtypeStruct((N_NODES, N_CLS), jnp.float32),
    )(X, W0, W1, W2)


R_BLK = 32


def _medoid_body(xg_ref, vals_ref, rs_ref, out_ref):
    xg = xg_ref[...].reshape(R_BLK, K, N_CLS)
    v = vals_ref[...]
    rs = rs_ref[...]
    sq = jnp.sum(xg * xg, axis=2)
    g = lax.dot_general(xg, xg, (((2,), (2,)), ((0,), (0,))),
                        preferred_element_type=jnp.float32)
    sqd = sq[:, :, None] + sq[:, None, :] - 2.0 * g
    l2 = jnp.sqrt(jnp.abs(sqd) + EPS)
    dist = jnp.sum(v[:, None, :] * l2, axis=2)
    dist = jnp.where(v == 0.0, FMAX, dist)
    m = jnp.max(-dist, axis=1, keepdims=True)
    e = jnp.exp(-dist - m)
    w = e / jnp.sum(e, axis=1, keepdims=True)
    w = w * v
    w = w / jnp.sum(w, axis=1, keepdims=True)
    out = lax.dot_general(w, xg, (((1,), (1,)), ((0,), (0,))),
                          preferred_element_type=jnp.float32)
    out_ref[...] = rs * out


def _medoid(xg, vals, rs_b, batch_s):
    return pl.pallas_call(
        _medoid_body,
        grid=(batch_s // R_BLK,),
        in_specs=[
            pl.BlockSpec((R_BLK * K, N_CLS), lambda i: (i, 0)),
            pl.BlockSpec((R_BLK, K), lambda i: (i, 0)),
            pl.BlockSpec((R_BLK, N_CLS), lambda i: (i, 0)),
        ],
        out_specs=pl.BlockSpec((R_BLK, N_CLS), lambda i: (i, 0)),
        out_shape=jax.ShapeDtypeStruct((batch_s, N_CLS), jnp.float32),
    )(xg, vals, rs_b)


N_PARTS = 2


def kernel(X, ppr_scores, W0, W1, W2):
    bs = BATCH // N_PARTS
    sc_topk, sc_gather = _sc_kernels(bs)
    logits = _mlp(X, W0, W1, W2)
    outs = []
    for p in range(N_PARTS):
        ppr_p = lax.slice_in_dim(ppr_scores, p * bs, (p + 1) * bs, axis=0)
        idx_flat, vals_flat, rs = sc_topk(ppr_p)
        xg = sc_gather(logits, idx_flat.reshape(bs * K // CH, CH))
        rs_b = jnp.broadcast_to(rs[:, None], (bs, N_CLS))
        outs.append(_medoid(xg, vals_flat.reshape(bs, K), rs_b, bs))
    return jnp.concatenate(outs, axis=0)


# speedup vs baseline: 1.0637x; 1.0637x over previous
import functools

import jax
import jax.numpy as jnp
from jax import lax
from jax.experimental import pallas as pl
from jax.experimental.pallas import tpu as pltpu
from jax.experimental.pallas import tpu_sc as plsc

N_NODES = 16384
D_FEAT = 512
HIDDEN = 1024
N_CLS = 128
BATCH = 2048
K = 64
EPS = 100.0 * float(jnp.finfo(jnp.float32).eps)
FMAX = float(jnp.finfo(jnp.float32).max)

NC, NS, L = 2, 16, 16
NW = NC * NS
ROWS_PER_W = BATCH // NW

CAP = 2048
GROUPS_PER_BLK = 16
BUFSZ = CAP + GROUPS_PER_BLK * L + L
N_BLKS = N_NODES // (GROUPS_PER_BLK * L)

T1 = 1.0 - 128.0 / N_NODES
T1_BITS = 0x3F7E0000
CAPF = 256
FGROUPS = (CAPF + 2 * L) // L
FBUF = 544
SENT = N_NODES
RSTRIDE = N_NODES + 128

def _sc_mesh():
    return plsc.VectorSubcoreMesh(
        core_axis_name="c", subcore_axis_name="s", num_cores=NC,
        num_subcores=NS)


def _iota16():
    return lax.iota(jnp.int32, L)


def _splat_i32(x):
    return jnp.full((L,), x, dtype=jnp.int32)


def _select_topk(rowv, base, cand, ptr):
    iota = _iota16()
    ptr_s = jnp.max(ptr)
    ngroups = (ptr_s + (L - 1)) // L

    def count_ge(tvec, strict):
        def body(g, cnt):
            bg = g * L
            idxg = cand[pl.ds(bg, L)]
            valid = (bg + iota) < ptr
            vg = plsc.load_gather(rowv, [idxg + base], mask=valid)
            hit = (vg > tvec if strict else vg >= tvec) & valid
            return cnt + plsc.all_reduce_population_count(hit)
        return lax.fori_loop(0, ngroups, body, _splat_i32(0))

    def bis_body(_, carry):
        lo, hi = carry
        mid = lax.shift_right_arithmetic(lo + hi, 1)
        cnt = count_ge(plsc.bitcast(mid, jnp.float32), False)
        ge = cnt >= K
        return jnp.where(ge, mid, lo), jnp.where(ge, hi, mid)

    lo, _ = lax.fori_loop(
        0, 30, bis_body, (_splat_i32(0), _splat_i32(0x3F800000)))
    tstar = plsc.bitcast(lo, jnp.float32)

    n_gt = count_ge(tstar, True)
    need = K - n_gt

    def compact(g, carry):
        wp, eqc = carry
        bg = g * L
        idxg = cand[pl.ds(bg, L)]
        valid = (bg + iota) < ptr
        vg = plsc.load_gather(rowv, [idxg + base], mask=valid)
        gt = (vg > tstar) & valid
        eq = (vg == tstar) & valid
        eq_rank = eqc + plsc.cumsum(eq.astype(jnp.int32))
        keep = gt | (eq & (eq_rank <= need))
        pos = wp + plsc.cumsum(keep.astype(jnp.int32)) - 1
        plsc.store_scatter(cand, [pos], idxg, mask=keep)
        return (wp + plsc.all_reduce_population_count(keep),
                eqc + plsc.all_reduce_population_count(eq))

    lax.fori_loop(0, ngroups, compact, (_splat_i32(0), _splat_i32(0)))
    return _splat_i32(K), tstar


def _sc_topk_body(rows_pw, ppr_hbm, idx_out, vals_out, rowv2, cand,
                  candi, candvv, actg, idx_st, vals_st, sem0, sem1):
    wid = lax.axis_index("s") * NC + lax.axis_index("c")
    row0 = wid * rows_pw
    iota = _iota16()
    sems = (sem0, sem1)
    for b in range(2):
        rowv2[pl.ds(b * RSTRIDE + SENT, L)] = jnp.full((L,), -1.0, jnp.float32)
    pltpu.async_copy(ppr_hbm.at[row0], rowv2.at[pl.ds(0, N_NODES)], sem0)

    def row_body(r, base):
        for j in range(FGROUPS):
            candi[pl.ds(j * L, L)] = _splat_i32(SENT)
            actg[pl.ds(j * L, L)] = _splat_i32(SENT // L)

        t1 = jnp.full((L,), T1, jnp.float32)

        def fblk(b, nact):
            cnts = jnp.zeros((L,), jnp.int32)
            for gg in range(GROUPS_PER_BLK):
                off = (b * GROUPS_PER_BLK + gg) * L
                v = rowv2[pl.ds(base + off, L)]
                pc = plsc.all_reduce_population_count(v > t1)
                cnts = jnp.where(iota == gg, pc, cnts)
            act = cnts > 0
            posn = nact + plsc.cumsum(act.astype(jnp.int32)) - 1
            posn = jnp.minimum(posn, FGROUPS * L - 1)
            plsc.store_scatter(actg, [posn], iota + b * L, mask=act)
            return nact + plsc.all_reduce_population_count(act)

        nact = lax.fori_loop(0, N_BLKS, fblk, _splat_i32(0))
        nact_s = jnp.minimum(jnp.max(nact), FGROUPS * L)

        def cblk(t, ptr):
            av = actg[pl.ds(t * L, L)]
            for j in range(L):
                gb = jnp.clip(av[j], 0, SENT // L) * L
                v = rowv2[pl.ds(base + gb, L)]
                m = v > t1
                pos = ptr + plsc.cumsum(m.astype(jnp.int32)) - 1
                pos = jnp.minimum(pos, FBUF - 1)
                plsc.store_scatter(candi, [pos], iota + gb, mask=m)
                ptr = ptr + plsc.all_reduce_population_count(m)
            return ptr

        ptr = lax.fori_loop(0, (nact_s + L - 1) // L, cblk, _splat_i32(0))
        ptr_s = jnp.max(ptr)
        ok = (ptr_s >= K) & (ptr_s <= CAPF) & (nact_s <= CAPF)

        @pl.when(ok)
        def _fast_select():
            for j in range(FGROUPS):
                ij = candi[pl.ds(j * L, L)]
                candvv[pl.ds(j * L, L)] = plsc.load_gather(rowv2, [ij + base])

            def count_s(tvec, strict):
                cnt = jnp.zeros((L,), jnp.int32)
                for j in range(FGROUPS):
                    vg = candvv[pl.ds(j * L, L)]
                    hit = vg > tvec if strict else vg >= tvec
                    cnt = cnt + hit.astype(jnp.int32)
                return jnp.sum(cnt, axis=0)

            def fbis(_, lohi):
                lo, hi = lohi
                mid = lax.shift_right_arithmetic(lo + hi, 1)
                tmid = lax.bitcast_convert_type(mid, jnp.float32)
                ge = count_s(jnp.full((L,), tmid), False) >= K
                return (jnp.where(ge, mid, lo), jnp.where(ge, hi, mid))

            lo, _ = lax.fori_loop(0, 17, fbis,
                                  (jnp.int32(T1_BITS), jnp.int32(0x3F800000)))
            tstar = jnp.full((L,), lax.bitcast_convert_type(lo, jnp.float32))
            need = K - _splat_i32(count_s(tstar, True))
            ngroups = (ptr_s + (L - 1)) // L

            def comp(g, carry):
                wp, eqc = carry
                base = g * L
                idxg = candi[pl.ds(base, L)]
                vg = candvv[pl.ds(base, L)]
                gt = vg > tstar
                eq = vg == tstar
                eq_rank = eqc + plsc.cumsum(eq.astype(jnp.int32))
                keep = gt | (eq & (eq_rank <= need))
                pos = wp + plsc.cumsum(keep.astype(jnp.int32)) - 1
                plsc.store_scatter(cand, [pos], idxg, mask=keep)
                return (wp + plsc.all_reduce_population_count(keep),
                        eqc + plsc.all_reduce_population_count(eq))

            lax.fori_loop(0, ngroups, comp,
                          (_splat_i32(0), _splat_i32(0)))

        @pl.when(jnp.logical_not(ok))
        def _fallback():
            def blk_body(b, carry):
                fptr, thr = carry
                for gg in range(GROUPS_PER_BLK):
                    off = (b * GROUPS_PER_BLK + gg) * L
                    v = rowv2[pl.ds(base + off, L)]
                    mask = v > thr
                    pos = fptr + plsc.cumsum(mask.astype(jnp.int32)) - 1
                    plsc.store_scatter(cand, [pos], iota + off, mask=mask)
                    fptr = fptr + plsc.all_reduce_population_count(mask)

                def rebuild():
                    return _select_topk(rowv2, base, cand, fptr)

                fptr, thr = lax.cond(jnp.max(fptr) > CAP, rebuild,
                                     lambda: (fptr, thr))
                return fptr, thr

            fptr, _ = lax.fori_loop(
                0, N_BLKS, blk_body,
                (_splat_i32(0), jnp.full((L,), -1.0, jnp.float32)))
            _select_topk(rowv2, base, cand, fptr)

        for j in range(K // L):
            idxg = cand[pl.ds(j * L, L)]
            vg = plsc.load_gather(rowv2, [idxg + base])
            idx_st[pl.ds(r * K + j * L, L)] = idxg
            vals_st[pl.ds(r * K + j * L, L)] = vg


    def pair(t, _):
        for b in range(2):
            r = t * 2 + b

            @pl.when(r + 1 < rows_pw)
            def _():
                pltpu.async_copy(
                    ppr_hbm.at[row0 + r + 1],
                    rowv2.at[pl.ds(((b + 1) % 2) * RSTRIDE, N_NODES)],
                    sems[(b + 1) % 2])

            pltpu.make_async_copy(ppr_hbm.at[row0 + r],
                                  rowv2.at[pl.ds(b * RSTRIDE, N_NODES)],
                                  sems[b]).wait()
            row_body(r, b * RSTRIDE)
        return 0

    lax.fori_loop(0, rows_pw // 2, pair, 0)
    pltpu.sync_copy(idx_st, idx_out.at[pl.ds(row0 * K, rows_pw * K)])
    pltpu.sync_copy(vals_st, vals_out.at[pl.ds(row0 * K, rows_pw * K)])


CH = 128
CHUNKS_PER_W = (BATCH * K) // NW // CH


def _sc_gather_body(chunks_pw, logits_hbm, idx_hbm, out_hbm, idx_v, rows_v,
                    sem0, sem1):
    wid = lax.axis_index("s") * NC + lax.axis_index("c")
    base = wid * chunks_pw * CH
    pltpu.sync_copy(idx_hbm.at[pl.ds(wid * chunks_pw, chunks_pw), :],
                    idx_v)
    sems = (sem0, sem1)
    pltpu.async_copy(logits_hbm.at[idx_v.at[0]], rows_v.at[0], sem0)

    def outer(t, _):
        for b in range(2):
            j = t * 2 + b

            @pl.when(j + 1 < chunks_pw)
            def _():
                pltpu.async_copy(logits_hbm.at[idx_v.at[j + 1]],
                                 rows_v.at[(b + 1) % 2], sems[(b + 1) % 2])

            pltpu.make_async_copy(logits_hbm.at[idx_v.at[j]], rows_v.at[b],
                                  sems[b]).wait()
            pltpu.sync_copy(rows_v.at[b],
                            out_hbm.at[pl.ds(base + j * CH, CH), :])
        return 0

    lax.fori_loop(0, chunks_pw // 2, outer, 0)


_SC_PARAMS = None


def _sc_params():
    return pltpu.CompilerParams(needs_layout_passes=False)


@functools.cache
def _sc_topk_kernel(batch_s):
    rows_pw = batch_s // NW
    return pl.kernel(
        functools.partial(_sc_topk_body, rows_pw),
        compiler_params=_sc_params(),
        out_type=(jax.ShapeDtypeStruct((batch_s * K,), jnp.int32),
                  jax.ShapeDtypeStruct((batch_s * K,), jnp.float32)),
        mesh=_sc_mesh(),
        scratch_types=[
            pltpu.VMEM((2 * RSTRIDE,), jnp.float32),
            pltpu.VMEM((BUFSZ,), jnp.int32),
            pltpu.VMEM((FBUF,), jnp.int32),
            pltpu.VMEM((FGROUPS * L,), jnp.float32),
            pltpu.VMEM((FGROUPS * L,), jnp.int32),
            pltpu.VMEM((rows_pw * K,), jnp.int32),
            pltpu.VMEM((rows_pw * K,), jnp.float32),
            pltpu.SemaphoreType.DMA,
            pltpu.SemaphoreType.DMA,
        ],
    )


@functools.cache
def _sc_gather_kernel(batch_s):
    chunks_pw = (batch_s * K) // NW // CH
    return pl.kernel(
        functools.partial(_sc_gather_body, chunks_pw),
        compiler_params=_sc_params(),
        out_type=jax.ShapeDtypeStruct((batch_s * K, N_CLS), jnp.float32),
        mesh=_sc_mesh(),
        scratch_types=[
            pltpu.VMEM((chunks_pw, CH), jnp.int32),
            pltpu.VMEM((2, CH, N_CLS), jnp.float32),
            pltpu.SemaphoreType.DMA,
            pltpu.SemaphoreType.DMA,
        ],
    )


RS_BLK = 128


def _rowsum_body(ppr_ref, out_ref):
    rs = jnp.sum(ppr_ref[...], axis=1, keepdims=True)
    out_ref[...] = jnp.broadcast_to(rs, (RS_BLK, N_CLS))


def _rowsum(ppr):
    return pl.pallas_call(
        _rowsum_body,
        grid=(BATCH // RS_BLK,),
        in_specs=[pl.BlockSpec((RS_BLK, N_NODES), lambda i: (i, 0))],
        out_specs=pl.BlockSpec((RS_BLK, N_CLS), lambda i: (i, 0)),
        out_shape=jax.ShapeDtypeStruct((BATCH, N_CLS), jnp.float32),
    )(ppr)


MLP_BLK = 512


def _mlp_body(x_ref, w0_ref, w1_ref, w2_ref, out_ref):
    h = jnp.dot(x_ref[...], w0_ref[...], preferred_element_type=jnp.float32)
    h = jnp.dot(jnp.maximum(h, 0.0), w1_ref[...],
                preferred_element_type=jnp.float32)
    out_ref[...] = jnp.dot(jnp.maximum(h, 0.0), w2_ref[...],
                           preferred_element_type=jnp.float32)


def _mlp(X, W0, W1, W2):
    return pl.pallas_call(
        _mlp_body,
        grid=(N_NODES // MLP_BLK,),
        in_specs=[
            pl.BlockSpec((MLP_BLK, D_FEAT), lambda i: (i, 0)),
            pl.BlockSpec((D_FEAT, HIDDEN), lambda i: (0, 0)),
            pl.BlockSpec((HIDDEN, HIDDEN), lambda i: (0, 0)),
            pl.BlockSpec((HIDDEN, N_CLS), lambda i: (0, 0)),
        ],
        out_specs=pl.BlockSpec((MLP_BLK, N_CLS), lambda i: (i, 0)),
        out_shape=jax.ShapeDtypeStruct((N_NODES, N_CLS), jnp.float32),
    )(X, W0, W1, W2)


R_BLK = 32


def _medoid_body(xg_ref, vals_ref, rs_ref, out_ref):
    xg = xg_ref[...].reshape(R_BLK, K, N_CLS)
    v = vals_ref[...]
    rs = rs_ref[...]
    sq = jnp.sum(xg * xg, axis=2)
    g = lax.dot_general(xg, xg, (((2,), (2,)), ((0,), (0,))),
                        preferred_element_type=jnp.float32)
    sqd = sq[:, :, None] + sq[:, None, :] - 2.0 * g
    l2 = jnp.sqrt(jnp.abs(sqd) + EPS)
    dist = jnp.sum(v[:, None, :] * l2, axis=2)
    dist = jnp.where(v == 0.0, FMAX, dist)
    m = jnp.max(-dist, axis=1, keepdims=True)
    e = jnp.exp(-dist - m)
    w = e / jnp.sum(e, axis=1, keepdims=True)
    w = w * v
    w = w / jnp.sum(w, axis=1, keepdims=True)
    out = lax.dot_general(w, xg, (((1,), (1,)), ((0,), (0,))),
                          preferred_element_type=jnp.float32)
    out_ref[...] = rs * out


def _medoid(xg, vals, rs_b, batch_s):
    return pl.pallas_call(
        _medoid_body,
        grid=(batch_s // R_BLK,),
        in_specs=[
            pl.BlockSpec((R_BLK * K, N_CLS), lambda i: (i, 0)),
            pl.BlockSpec((R_BLK, K), lambda i: (i, 0)),
            pl.BlockSpec((R_BLK, N_CLS), lambda i: (i, 0)),
        ],
        out_specs=pl.BlockSpec((R_BLK, N_CLS), lambda i: (i, 0)),
        out_shape=jax.ShapeDtypeStruct((batch_s, N_CLS), jnp.float32),
    )(xg, vals, rs_b)


N_PARTS = 2


def kernel(X, ppr_scores, W0, W1, W2):
    bs = BATCH // N_PARTS
    logits = _mlp(X, W0, W1, W2)
    rs_b = _rowsum(ppr_scores)
    idx_flat, vals_flat = _sc_topk_kernel(BATCH)(ppr_scores)
    idx2d = idx_flat.reshape(BATCH * K // CH, CH)
    vals2d = vals_flat.reshape(BATCH, K)
    rows_per_part = bs * K // CH
    outs = []
    for p in range(N_PARTS):
        idx_p = lax.slice_in_dim(idx2d, p * rows_per_part,
                                 (p + 1) * rows_per_part, axis=0)
        xg = _sc_gather_kernel(bs)(logits, idx_p)
        vals_p = lax.slice_in_dim(vals2d, p * bs, (p + 1) * bs, axis=0)
        rs_p = lax.slice_in_dim(rs_b, p * bs, (p + 1) * bs, axis=0)
        outs.append(_medoid(xg, vals_p, rs_p, bs))
    return jnp.concatenate(outs, axis=0)


# speedup vs baseline: 1.2091x; 1.1367x over previous
import functools

import jax
import jax.numpy as jnp
from jax import lax
from jax.experimental import pallas as pl
from jax.experimental.pallas import tpu as pltpu
from jax.experimental.pallas import tpu_sc as plsc

N_NODES = 16384
D_FEAT = 512
HIDDEN = 1024
N_CLS = 128
BATCH = 2048
K = 64
EPS = 100.0 * float(jnp.finfo(jnp.float32).eps)
FMAX = float(jnp.finfo(jnp.float32).max)

NC, NS, L = 2, 16, 16
NW = NC * NS
ROWS_PER_W = BATCH // NW

CAP = 2048
GROUPS_PER_BLK = 16
BUFSZ = CAP + GROUPS_PER_BLK * L + L
N_BLKS = N_NODES // (GROUPS_PER_BLK * L)

T1 = 1.0 - 128.0 / N_NODES
T1_BITS = 0x3F7E0000
CAPF = 256
FGROUPS = (CAPF + 2 * L) // L
FBUF = 544
SENT = N_NODES
RSTRIDE = N_NODES + 128

def _sc_mesh():
    return plsc.VectorSubcoreMesh(
        core_axis_name="c", subcore_axis_name="s", num_cores=NC,
        num_subcores=NS)


def _iota16():
    return lax.iota(jnp.int32, L)


def _splat_i32(x):
    return jnp.full((L,), x, dtype=jnp.int32)


def _select_topk(rowv, base, cand, ptr):
    iota = _iota16()
    ptr_s = jnp.max(ptr)
    ngroups = (ptr_s + (L - 1)) // L

    def count_ge(tvec, strict):
        def body(g, cnt):
            bg = g * L
            idxg = cand[pl.ds(bg, L)]
            valid = (bg + iota) < ptr
            vg = plsc.load_gather(rowv, [idxg + base], mask=valid)
            hit = (vg > tvec if strict else vg >= tvec) & valid
            return cnt + plsc.all_reduce_population_count(hit)
        return lax.fori_loop(0, ngroups, body, _splat_i32(0))

    def bis_body(_, carry):
        lo, hi = carry
        mid = lax.shift_right_arithmetic(lo + hi, 1)
        cnt = count_ge(plsc.bitcast(mid, jnp.float32), False)
        ge = cnt >= K
        return jnp.where(ge, mid, lo), jnp.where(ge, hi, mid)

    lo, _ = lax.fori_loop(
        0, 30, bis_body, (_splat_i32(0), _splat_i32(0x3F800000)))
    tstar = plsc.bitcast(lo, jnp.float32)

    n_gt = count_ge(tstar, True)
    need = K - n_gt

    def compact(g, carry):
        wp, eqc = carry
        bg = g * L
        idxg = cand[pl.ds(bg, L)]
        valid = (bg + iota) < ptr
        vg = plsc.load_gather(rowv, [idxg + base], mask=valid)
        gt = (vg > tstar) & valid
        eq = (vg == tstar) & valid
        eq_rank = eqc + plsc.cumsum(eq.astype(jnp.int32))
        keep = gt | (eq & (eq_rank <= need))
        pos = wp + plsc.cumsum(keep.astype(jnp.int32)) - 1
        plsc.store_scatter(cand, [pos], idxg, mask=keep)
        return (wp + plsc.all_reduce_population_count(keep),
                eqc + plsc.all_reduce_population_count(eq))

    lax.fori_loop(0, ngroups, compact, (_splat_i32(0), _splat_i32(0)))
    return _splat_i32(K), tstar


def _sc_topk_body(rows_pw, ppr_hbm, idx_out, vals_out, rs_out, rowv2, cand,
                  candi, candvv, actg, idx_st, vals_st, rs_st, sem0, sem1):
    wid = lax.axis_index("s") * NC + lax.axis_index("c")
    row0 = wid * rows_pw
    iota = _iota16()
    lane0 = iota == 0
    sems = (sem0, sem1)
    for b in range(2):
        rowv2[pl.ds(b * RSTRIDE + SENT, L)] = jnp.full((L,), -1.0, jnp.float32)
    pltpu.async_copy(ppr_hbm.at[row0], rowv2.at[pl.ds(0, N_NODES)], sem0)

    def row_body(r, base):
        for j in range(FGROUPS):
            candi[pl.ds(j * L, L)] = _splat_i32(SENT)
            actg[pl.ds(j * L, L)] = _splat_i32(SENT // L)

        t1 = jnp.full((L,), T1, jnp.float32)

        def fblk(b, carry):
            acc0, acc1, nact = carry
            cnts = jnp.zeros((L,), jnp.int32)
            for gg in range(GROUPS_PER_BLK):
                off = (b * GROUPS_PER_BLK + gg) * L
                v = rowv2[pl.ds(base + off, L)]
                if gg % 2 == 0:
                    acc0 = acc0 + v
                else:
                    acc1 = acc1 + v
                pc = plsc.all_reduce_population_count(v > t1)
                cnts = jnp.where(iota == gg, pc, cnts)
            act = cnts > 0
            posn = nact + plsc.cumsum(act.astype(jnp.int32)) - 1
            posn = jnp.minimum(posn, FGROUPS * L - 1)
            plsc.store_scatter(actg, [posn], iota + b * L, mask=act)
            return acc0, acc1, nact + plsc.all_reduce_population_count(act)

        acc0, acc1, nact = lax.fori_loop(
            0, N_BLKS, fblk,
            (jnp.zeros((L,), jnp.float32), jnp.zeros((L,), jnp.float32),
             _splat_i32(0)))
        acc = acc0 + acc1
        nact_s = jnp.minimum(jnp.max(nact), FGROUPS * L)

        def cblk(t, ptr):
            av = actg[pl.ds(t * L, L)]
            for j in range(L):
                gb = jnp.clip(av[j], 0, SENT // L) * L
                v = rowv2[pl.ds(base + gb, L)]
                m = v > t1
                pos = ptr + plsc.cumsum(m.astype(jnp.int32)) - 1
                pos = jnp.minimum(pos, FBUF - 1)
                plsc.store_scatter(candi, [pos], iota + gb, mask=m)
                ptr = ptr + plsc.all_reduce_population_count(m)
            return ptr

        ptr = lax.fori_loop(0, (nact_s + L - 1) // L, cblk, _splat_i32(0))
        ptr_s = jnp.max(ptr)
        ok = (ptr_s >= K) & (ptr_s <= CAPF) & (nact_s <= CAPF)

        @pl.when(ok)
        def _fast_select():
            for j in range(FGROUPS):
                ij = candi[pl.ds(j * L, L)]
                candvv[pl.ds(j * L, L)] = plsc.load_gather(rowv2, [ij + base])

            def count_s(tvec, strict):
                cnt = jnp.zeros((L,), jnp.int32)
                for j in range(FGROUPS):
                    vg = candvv[pl.ds(j * L, L)]
                    hit = vg > tvec if strict else vg >= tvec
                    cnt = cnt + hit.astype(jnp.int32)
                return jnp.sum(cnt, axis=0)

            def fbis(_, lohi):
                lo, hi = lohi
                mid = lax.shift_right_arithmetic(lo + hi, 1)
                tmid = lax.bitcast_convert_type(mid, jnp.float32)
                ge = count_s(jnp.full((L,), tmid), False) >= K
                return (jnp.where(ge, mid, lo), jnp.where(ge, hi, mid))

            lo, _ = lax.fori_loop(0, 17, fbis,
                                  (jnp.int32(T1_BITS), jnp.int32(0x3F800000)))
            tstar = jnp.full((L,), lax.bitcast_convert_type(lo, jnp.float32))
            need = K - _splat_i32(count_s(tstar, True))
            ngroups = (ptr_s + (L - 1)) // L

            def comp(g, carry):
                wp, eqc = carry
                base = g * L
                idxg = candi[pl.ds(base, L)]
                vg = candvv[pl.ds(base, L)]
                gt = vg > tstar
                eq = vg == tstar
                eq_rank = eqc + plsc.cumsum(eq.astype(jnp.int32))
                keep = gt | (eq & (eq_rank <= need))
                pos = wp + plsc.cumsum(keep.astype(jnp.int32)) - 1
                plsc.store_scatter(cand, [pos], idxg, mask=keep)
                return (wp + plsc.all_reduce_population_count(keep),
                        eqc + plsc.all_reduce_population_count(eq))

            lax.fori_loop(0, ngroups, comp,
                          (_splat_i32(0), _splat_i32(0)))

        @pl.when(jnp.logical_not(ok))
        def _fallback():
            def blk_body(b, carry):
                fptr, thr = carry
                for gg in range(GROUPS_PER_BLK):
                    off = (b * GROUPS_PER_BLK + gg) * L
                    v = rowv2[pl.ds(base + off, L)]
                    mask = v > thr
                    pos = fptr + plsc.cumsum(mask.astype(jnp.int32)) - 1
                    plsc.store_scatter(cand, [pos], iota + off, mask=mask)
                    fptr = fptr + plsc.all_reduce_population_count(mask)

                def rebuild():
                    return _select_topk(rowv2, base, cand, fptr)

                fptr, thr = lax.cond(jnp.max(fptr) > CAP, rebuild,
                                     lambda: (fptr, thr))
                return fptr, thr

            fptr, _ = lax.fori_loop(
                0, N_BLKS, blk_body,
                (_splat_i32(0), jnp.full((L,), -1.0, jnp.float32)))
            _select_topk(rowv2, base, cand, fptr)

        for j in range(K // L):
            idxg = cand[pl.ds(j * L, L)]
            vg = plsc.load_gather(rowv2, [idxg + base])
            idx_st[pl.ds(r * K + j * L, L)] = idxg
            vals_st[pl.ds(r * K + j * L, L)] = vg

        rsum = jnp.sum(acc, axis=0)
        plsc.store_scatter(rs_st, [_splat_i32(r)],
                           jnp.full((L,), rsum, jnp.float32), mask=lane0)


    def pair(t, _):
        for b in range(2):
            r = t * 2 + b

            @pl.when(r + 1 < rows_pw)
            def _():
                pltpu.async_copy(
                    ppr_hbm.at[row0 + r + 1],
                    rowv2.at[pl.ds(((b + 1) % 2) * RSTRIDE, N_NODES)],
                    sems[(b + 1) % 2])

            pltpu.make_async_copy(ppr_hbm.at[row0 + r],
                                  rowv2.at[pl.ds(b * RSTRIDE, N_NODES)],
                                  sems[b]).wait()
            row_body(r, b * RSTRIDE)
        return 0

    lax.fori_loop(0, rows_pw // 2, pair, 0)
    pltpu.sync_copy(idx_st, idx_out.at[pl.ds(row0 * K, rows_pw * K)])
    pltpu.sync_copy(vals_st, vals_out.at[pl.ds(row0 * K, rows_pw * K)])
    pltpu.sync_copy(rs_st, rs_out.at[pl.ds(row0, rows_pw)])


CH = 128
CHUNKS_PER_W = (BATCH * K) // NW // CH


def _sc_gather_body(chunks_pw, logits_hbm, idx_hbm, out_hbm, idx_v, rows_v,
                    sem0, sem1):
    wid = lax.axis_index("s") * NC + lax.axis_index("c")
    base = wid * chunks_pw * CH
    pltpu.sync_copy(idx_hbm.at[pl.ds(wid * chunks_pw, chunks_pw), :],
                    idx_v)
    sems = (sem0, sem1)
    pltpu.async_copy(logits_hbm.at[idx_v.at[0]], rows_v.at[0], sem0)

    def outer(t, _):
        for b in range(2):
            j = t * 2 + b

            @pl.when(j + 1 < chunks_pw)
            def _():
                pltpu.async_copy(logits_hbm.at[idx_v.at[j + 1]],
                                 rows_v.at[(b + 1) % 2], sems[(b + 1) % 2])

            pltpu.make_async_copy(logits_hbm.at[idx_v.at[j]], rows_v.at[b],
                                  sems[b]).wait()
            pltpu.sync_copy(rows_v.at[b],
                            out_hbm.at[pl.ds(base + j * CH, CH), :])
        return 0

    lax.fori_loop(0, chunks_pw // 2, outer, 0)


_SC_PARAMS = None


def _sc_params():
    return pltpu.CompilerParams(needs_layout_passes=False)


@functools.cache
def _sc_topk_kernel(batch_s):
    rows_pw = batch_s // NW
    return pl.kernel(
        functools.partial(_sc_topk_body, rows_pw),
        compiler_params=_sc_params(),
        out_type=(jax.ShapeDtypeStruct((batch_s * K,), jnp.int32),
                  jax.ShapeDtypeStruct((batch_s * K,), jnp.float32),
                  jax.ShapeDtypeStruct((batch_s,), jnp.float32)),
        mesh=_sc_mesh(),
        scratch_types=[
            pltpu.VMEM((2 * RSTRIDE,), jnp.float32),
            pltpu.VMEM((BUFSZ,), jnp.int32),
            pltpu.VMEM((FBUF,), jnp.int32),
            pltpu.VMEM((FGROUPS * L,), jnp.float32),
            pltpu.VMEM((FGROUPS * L,), jnp.int32),
            pltpu.VMEM((rows_pw * K,), jnp.int32),
            pltpu.VMEM((rows_pw * K,), jnp.float32),
            pltpu.VMEM((rows_pw,), jnp.float32),
            pltpu.SemaphoreType.DMA,
            pltpu.SemaphoreType.DMA,
        ],
    )


@functools.cache
def _sc_gather_kernel(batch_s):
    chunks_pw = (batch_s * K) // NW // CH
    return pl.kernel(
        functools.partial(_sc_gather_body, chunks_pw),
        compiler_params=_sc_params(),
        out_type=jax.ShapeDtypeStruct((batch_s * K, N_CLS), jnp.float32),
        mesh=_sc_mesh(),
        scratch_types=[
            pltpu.VMEM((chunks_pw, CH), jnp.int32),
            pltpu.VMEM((2, CH, N_CLS), jnp.float32),
            pltpu.SemaphoreType.DMA,
            pltpu.SemaphoreType.DMA,
        ],
    )


MLP_BLK = 512


def _mlp_body(x_ref, w0_ref, w1_ref, w2_ref, out_ref):
    h = jnp.dot(x_ref[...], w0_ref[...], preferred_element_type=jnp.float32)
    h = jnp.dot(jnp.maximum(h, 0.0), w1_ref[...],
                preferred_element_type=jnp.float32)
    out_ref[...] = jnp.dot(jnp.maximum(h, 0.0), w2_ref[...],
                           preferred_element_type=jnp.float32)


def _mlp(X, W0, W1, W2):
    return pl.pallas_call(
        _mlp_body,
        grid=(N_NODES // MLP_BLK,),
        in_specs=[
            pl.BlockSpec((MLP_BLK, D_FEAT), lambda i: (i, 0)),
            pl.BlockSpec((D_FEAT, HIDDEN), lambda i: (0, 0)),
            pl.BlockSpec((HIDDEN, HIDDEN), lambda i: (0, 0)),
            pl.BlockSpec((HIDDEN, N_CLS), lambda i: (0, 0)),
        ],
        out_specs=pl.BlockSpec((MLP_BLK, N_CLS), lambda i: (i, 0)),
        out_shape=jax.ShapeDtypeStruct((N_NODES, N_CLS), jnp.float32),
    )(X, W0, W1, W2)


R_BLK = 32


def _medoid_body(xg_ref, vals_ref, rs_ref, out_ref):
    xg = xg_ref[...].reshape(R_BLK, K, N_CLS)
    v = vals_ref[...]
    rs = rs_ref[...]
    sq = jnp.sum(xg * xg, axis=2)
    g = lax.dot_general(xg, xg, (((2,), (2,)), ((0,), (0,))),
                        preferred_element_type=jnp.float32)
    sqd = sq[:, :, None] + sq[:, None, :] - 2.0 * g
    l2 = jnp.sqrt(jnp.abs(sqd) + EPS)
    dist = jnp.sum(v[:, None, :] * l2, axis=2)
    dist = jnp.where(v == 0.0, FMAX, dist)
    m = jnp.max(-dist, axis=1, keepdims=True)
    e = jnp.exp(-dist - m)
    w = e / jnp.sum(e, axis=1, keepdims=True)
    w = w * v
    w = w / jnp.sum(w, axis=1, keepdims=True)
    out = lax.dot_general(w, xg, (((1,), (1,)), ((0,), (0,))),
                          preferred_element_type=jnp.float32)
    out_ref[...] = rs * out


def _medoid(xg, vals, rs_b, batch_s):
    return pl.pallas_call(
        _medoid_body,
        grid=(batch_s // R_BLK,),
        in_specs=[
            pl.BlockSpec((R_BLK * K, N_CLS), lambda i: (i, 0)),
            pl.BlockSpec((R_BLK, K), lambda i: (i, 0)),
            pl.BlockSpec((R_BLK, N_CLS), lambda i: (i, 0)),
        ],
        out_specs=pl.BlockSpec((R_BLK, N_CLS), lambda i: (i, 0)),
        out_shape=jax.ShapeDtypeStruct((batch_s, N_CLS), jnp.float32),
    )(xg, vals, rs_b)


N_PARTS = 2


def kernel(X, ppr_scores, W0, W1, W2):
    bs = BATCH // N_PARTS
    logits = _mlp(X, W0, W1, W2)
    idx_flat, vals_flat, rs = _sc_topk_kernel(BATCH)(ppr_scores)
    rs_b = jnp.broadcast_to(rs[:, None], (BATCH, N_CLS))
    idx2d = idx_flat.reshape(BATCH * K // CH, CH)
    vals2d = vals_flat.reshape(BATCH, K)
    rows_per_part = bs * K // CH
    outs = []
    for p in range(N_PARTS):
        idx_p = lax.slice_in_dim(idx2d, p * rows_per_part,
                                 (p + 1) * rows_per_part, axis=0)
        xg = _sc_gather_kernel(bs)(logits, idx_p)
        vals_p = lax.slice_in_dim(vals2d, p * bs, (p + 1) * bs, axis=0)
        rs_p = lax.slice_in_dim(rs_b, p * bs, (p + 1) * bs, axis=0)
        outs.append(_medoid(xg, vals_p, rs_p, bs))
    return jnp.concatenate(outs, axis=0)


# speedup vs baseline: 1.2679x; 1.0486x over previous
import functools

import jax
import jax.numpy as jnp
from jax import lax
from jax.experimental import pallas as pl
from jax.experimental.pallas import tpu as pltpu
from jax.experimental.pallas import tpu_sc as plsc

N_NODES = 16384
D_FEAT = 512
HIDDEN = 1024
N_CLS = 128
BATCH = 2048
K = 64
EPS = 100.0 * float(jnp.finfo(jnp.float32).eps)
FMAX = float(jnp.finfo(jnp.float32).max)

NC, NS, L = 2, 16, 16
NW = NC * NS
ROWS_PER_W = BATCH // NW

CAP = 2048
GROUPS_PER_BLK = 16
BUFSZ = CAP + GROUPS_PER_BLK * L + L
N_BLKS = N_NODES // (GROUPS_PER_BLK * L)

T1 = 1.0 - 128.0 / N_NODES
T1_BITS = 0x3F7E0000
CAPF = 256
FGROUPS = (CAPF + 2 * L) // L
FBUF = 544
SENT = N_NODES
RSTRIDE = N_NODES + 128

def _sc_mesh():
    return plsc.VectorSubcoreMesh(
        core_axis_name="c", subcore_axis_name="s", num_cores=NC,
        num_subcores=NS)


def _iota16():
    return lax.iota(jnp.int32, L)


def _splat_i32(x):
    return jnp.full((L,), x, dtype=jnp.int32)


def _select_topk(rowv, base, cand, ptr):
    iota = _iota16()
    ptr_s = jnp.max(ptr)
    ngroups = (ptr_s + (L - 1)) // L

    def count_ge(tvec, strict):
        def body(g, cnt):
            bg = g * L
            idxg = cand[pl.ds(bg, L)]
            valid = (bg + iota) < ptr
            vg = plsc.load_gather(rowv, [idxg + base], mask=valid)
            hit = (vg > tvec if strict else vg >= tvec) & valid
            return cnt + plsc.all_reduce_population_count(hit)
        return lax.fori_loop(0, ngroups, body, _splat_i32(0))

    def bis_body(_, carry):
        lo, hi = carry
        mid = lax.shift_right_arithmetic(lo + hi, 1)
        cnt = count_ge(plsc.bitcast(mid, jnp.float32), False)
        ge = cnt >= K
        return jnp.where(ge, mid, lo), jnp.where(ge, hi, mid)

    lo, _ = lax.fori_loop(
        0, 30, bis_body, (_splat_i32(0), _splat_i32(0x3F800000)))
    tstar = plsc.bitcast(lo, jnp.float32)

    n_gt = count_ge(tstar, True)
    need = K - n_gt

    def compact(g, carry):
        wp, eqc = carry
        bg = g * L
        idxg = cand[pl.ds(bg, L)]
        valid = (bg + iota) < ptr
        vg = plsc.load_gather(rowv, [idxg + base], mask=valid)
        gt = (vg > tstar) & valid
        eq = (vg == tstar) & valid
        eq_rank = eqc + plsc.cumsum(eq.astype(jnp.int32))
        keep = gt | (eq & (eq_rank <= need))
        pos = wp + plsc.cumsum(keep.astype(jnp.int32)) - 1
        plsc.store_scatter(cand, [pos], idxg, mask=keep)
        return (wp + plsc.all_reduce_population_count(keep),
                eqc + plsc.all_reduce_population_count(eq))

    lax.fori_loop(0, ngroups, compact, (_splat_i32(0), _splat_i32(0)))
    return _splat_i32(K), tstar


def _sc_topk_body(rows_pw, ppr_hbm, idx_out, vals_out, rs_out, rowv2, cand,
                  candi, candvv, actg, idx_st, vals_st, rs_st, sem0, sem1):
    wid = lax.axis_index("s") * NC + lax.axis_index("c")
    row0 = wid * rows_pw
    iota = _iota16()
    lane0 = iota == 0
    sems = (sem0, sem1)
    for b in range(2):
        rowv2[pl.ds(b * RSTRIDE + SENT, L)] = jnp.full((L,), -1.0, jnp.float32)
    pltpu.async_copy(ppr_hbm.at[row0], rowv2.at[pl.ds(0, N_NODES)], sem0)

    def row_body(r, base):
        for j in range(FGROUPS):
            candi[pl.ds(j * L, L)] = _splat_i32(SENT)
            actg[pl.ds(j * L, L)] = _splat_i32(SENT // L)

        t1 = jnp.full((L,), T1, jnp.float32)

        def fblk(b, carry):
            acc0, acc1, nact = carry
            cnts = jnp.zeros((L,), jnp.int32)
            for gg in range(GROUPS_PER_BLK):
                off = (b * GROUPS_PER_BLK + gg) * L
                v = rowv2[pl.ds(base + off, L)]
                if gg % 2 == 0:
                    acc0 = acc0 + v
                else:
                    acc1 = acc1 + v
                pc = plsc.all_reduce_population_count(v > t1)
                cnts = jnp.where(iota == gg, pc, cnts)
            act = cnts > 0
            posn = nact + plsc.cumsum(act.astype(jnp.int32)) - 1
            posn = jnp.minimum(posn, FGROUPS * L - 1)
            plsc.store_scatter(actg, [posn], iota + b * L, mask=act)
            return acc0, acc1, nact + plsc.all_reduce_population_count(act)

        acc0, acc1, nact = lax.fori_loop(
            0, N_BLKS, fblk,
            (jnp.zeros((L,), jnp.float32), jnp.zeros((L,), jnp.float32),
             _splat_i32(0)))
        acc = acc0 + acc1
        nact_s = jnp.minimum(jnp.max(nact), FGROUPS * L)

        def cblk(t, ptr):
            av = actg[pl.ds(t * L, L)]
            for j in range(L):
                gb = jnp.clip(av[j], 0, SENT // L) * L
                v = rowv2[pl.ds(base + gb, L)]
                m = v > t1
                pos = ptr + plsc.cumsum(m.astype(jnp.int32)) - 1
                pos = jnp.minimum(pos, FBUF - 1)
                plsc.store_scatter(candi, [pos], iota + gb, mask=m)
                ptr = ptr + plsc.all_reduce_population_count(m)
            return ptr

        ptr = lax.fori_loop(0, (nact_s + L - 1) // L, cblk, _splat_i32(0))
        ptr_s = jnp.max(ptr)
        ok = (ptr_s >= K) & (ptr_s <= CAPF) & (nact_s <= CAPF)

        @pl.when(ok)
        def _fast_select():
            for j in range(FGROUPS):
                ij = candi[pl.ds(j * L, L)]
                candvv[pl.ds(j * L, L)] = plsc.load_gather(rowv2, [ij + base])

            def count_s(tvec, strict):
                cnt = jnp.zeros((L,), jnp.int32)
                for j in range(FGROUPS):
                    vg = candvv[pl.ds(j * L, L)]
                    hit = vg > tvec if strict else vg >= tvec
                    cnt = cnt + hit.astype(jnp.int32)
                return jnp.sum(cnt, axis=0)

            def fbis(_, lohi):
                lo, hi = lohi
                mid = lax.shift_right_arithmetic(lo + hi, 1)
                tmid = lax.bitcast_convert_type(mid, jnp.float32)
                ge = count_s(jnp.full((L,), tmid), False) >= K
                return (jnp.where(ge, mid, lo), jnp.where(ge, hi, mid))

            lo, _ = lax.fori_loop(0, 17, fbis,
                                  (jnp.int32(T1_BITS), jnp.int32(0x3F800000)))
            tstar = jnp.full((L,), lax.bitcast_convert_type(lo, jnp.float32))
            need = K - _splat_i32(count_s(tstar, True))
            ngroups = (ptr_s + (L - 1)) // L

            def comp(g, carry):
                wp, eqc = carry
                base = g * L
                idxg = candi[pl.ds(base, L)]
                vg = candvv[pl.ds(base, L)]
                gt = vg > tstar
                eq = vg == tstar
                eq_rank = eqc + plsc.cumsum(eq.astype(jnp.int32))
                keep = gt | (eq & (eq_rank <= need))
                pos = wp + plsc.cumsum(keep.astype(jnp.int32)) - 1
                plsc.store_scatter(cand, [pos], idxg, mask=keep)
                return (wp + plsc.all_reduce_population_count(keep),
                        eqc + plsc.all_reduce_population_count(eq))

            lax.fori_loop(0, ngroups, comp,
                          (_splat_i32(0), _splat_i32(0)))

        @pl.when(jnp.logical_not(ok))
        def _fallback():
            def blk_body(b, carry):
                fptr, thr = carry
                for gg in range(GROUPS_PER_BLK):
                    off = (b * GROUPS_PER_BLK + gg) * L
                    v = rowv2[pl.ds(base + off, L)]
                    mask = v > thr
                    pos = fptr + plsc.cumsum(mask.astype(jnp.int32)) - 1
                    plsc.store_scatter(cand, [pos], iota + off, mask=mask)
                    fptr = fptr + plsc.all_reduce_population_count(mask)

                def rebuild():
                    return _select_topk(rowv2, base, cand, fptr)

                fptr, thr = lax.cond(jnp.max(fptr) > CAP, rebuild,
                                     lambda: (fptr, thr))
                return fptr, thr

            fptr, _ = lax.fori_loop(
                0, N_BLKS, blk_body,
                (_splat_i32(0), jnp.full((L,), -1.0, jnp.float32)))
            _select_topk(rowv2, base, cand, fptr)

        for j in range(K // L):
            idxg = cand[pl.ds(j * L, L)]
            vg = plsc.load_gather(rowv2, [idxg + base])
            idx_st[pl.ds(r * K + j * L, L)] = idxg
            vals_st[pl.ds(r * K + j * L, L)] = vg

        rsum = jnp.sum(acc, axis=0)
        plsc.store_scatter(rs_st, [_splat_i32(r)],
                           jnp.full((L,), rsum, jnp.float32), mask=lane0)


    def pair(t, _):
        for b in range(2):
            r = t * 2 + b

            @pl.when(r + 1 < rows_pw)
            def _():
                pltpu.async_copy(
                    ppr_hbm.at[row0 + r + 1],
                    rowv2.at[pl.ds(((b + 1) % 2) * RSTRIDE, N_NODES)],
                    sems[(b + 1) % 2])

            pltpu.make_async_copy(ppr_hbm.at[row0 + r],
                                  rowv2.at[pl.ds(b * RSTRIDE, N_NODES)],
                                  sems[b]).wait()
            row_body(r, b * RSTRIDE)
        return 0

    lax.fori_loop(0, rows_pw // 2, pair, 0)
    pltpu.sync_copy(idx_st, idx_out.at[pl.ds(row0 * K, rows_pw * K)])
    pltpu.sync_copy(vals_st, vals_out.at[pl.ds(row0 * K, rows_pw * K)])
    pltpu.sync_copy(rs_st, rs_out.at[pl.ds(row0, rows_pw)])


CH = 128
CHUNKS_PER_W = (BATCH * K) // NW // CH


def _sc_gather_body(chunks_pw, logits_hbm, idx_hbm, out_hbm, idx_v, rows_v,
                    sem0, sem1):
    wid = lax.axis_index("s") * NC + lax.axis_index("c")
    base = wid * chunks_pw * CH
    pltpu.sync_copy(idx_hbm.at[pl.ds(wid * chunks_pw, chunks_pw), :],
                    idx_v)
    sems = (sem0, sem1)
    pltpu.async_copy(logits_hbm.at[idx_v.at[0]], rows_v.at[0], sem0)

    def outer(t, _):
        for b in range(2):
            j = t * 2 + b

            @pl.when(j + 1 < chunks_pw)
            def _():
                pltpu.async_copy(logits_hbm.at[idx_v.at[j + 1]],
                                 rows_v.at[(b + 1) % 2], sems[(b + 1) % 2])

            pltpu.make_async_copy(logits_hbm.at[idx_v.at[j]], rows_v.at[b],
                                  sems[b]).wait()
            pltpu.sync_copy(rows_v.at[b],
                            out_hbm.at[pl.ds(base + j * CH, CH), :])
        return 0

    lax.fori_loop(0, chunks_pw // 2, outer, 0)


_SC_PARAMS = None


def _sc_params():
    return pltpu.CompilerParams(needs_layout_passes=False)


@functools.cache
def _sc_topk_kernel(batch_s):
    rows_pw = batch_s // NW
    return pl.kernel(
        functools.partial(_sc_topk_body, rows_pw),
        compiler_params=_sc_params(),
        out_type=(jax.ShapeDtypeStruct((batch_s * K,), jnp.int32),
                  jax.ShapeDtypeStruct((batch_s * K,), jnp.float32),
                  jax.ShapeDtypeStruct((batch_s,), jnp.float32)),
        mesh=_sc_mesh(),
        scratch_types=[
            pltpu.VMEM((2 * RSTRIDE,), jnp.float32),
            pltpu.VMEM((BUFSZ,), jnp.int32),
            pltpu.VMEM((FBUF,), jnp.int32),
            pltpu.VMEM((FGROUPS * L,), jnp.float32),
            pltpu.VMEM((FGROUPS * L,), jnp.int32),
            pltpu.VMEM((rows_pw * K,), jnp.int32),
            pltpu.VMEM((rows_pw * K,), jnp.float32),
            pltpu.VMEM((rows_pw,), jnp.float32),
            pltpu.SemaphoreType.DMA,
            pltpu.SemaphoreType.DMA,
        ],
    )


@functools.cache
def _sc_gather_kernel(batch_s):
    chunks_pw = (batch_s * K) // NW // CH
    return pl.kernel(
        functools.partial(_sc_gather_body, chunks_pw),
        compiler_params=_sc_params(),
        out_type=jax.ShapeDtypeStruct((batch_s * K, N_CLS), jnp.float32),
        mesh=_sc_mesh(),
        scratch_types=[
            pltpu.VMEM((chunks_pw, CH), jnp.int32),
            pltpu.VMEM((2, CH, N_CLS), jnp.float32),
            pltpu.SemaphoreType.DMA,
            pltpu.SemaphoreType.DMA,
        ],
    )


MLP_BLK = 512


def _mlp_body(x_ref, w0_ref, w1_ref, w2_ref, out_ref):
    h = jnp.dot(x_ref[...], w0_ref[...], preferred_element_type=jnp.float32)
    h = jnp.dot(jnp.maximum(h, 0.0), w1_ref[...],
                preferred_element_type=jnp.float32)
    out_ref[...] = jnp.dot(jnp.maximum(h, 0.0), w2_ref[...],
                           preferred_element_type=jnp.float32)


def _mlp(X, W0, W1, W2):
    return pl.pallas_call(
        _mlp_body,
        grid=(N_NODES // MLP_BLK,),
        in_specs=[
            pl.BlockSpec((MLP_BLK, D_FEAT), lambda i: (i, 0)),
            pl.BlockSpec((D_FEAT, HIDDEN), lambda i: (0, 0)),
            pl.BlockSpec((HIDDEN, HIDDEN), lambda i: (0, 0)),
            pl.BlockSpec((HIDDEN, N_CLS), lambda i: (0, 0)),
        ],
        out_specs=pl.BlockSpec((MLP_BLK, N_CLS), lambda i: (i, 0)),
        out_shape=jax.ShapeDtypeStruct((N_NODES, N_CLS), jnp.float32),
    )(X, W0, W1, W2)


R_BLK = 64


def _medoid_body(xg_ref, vals_ref, rs_ref, out_ref):
    xg = xg_ref[...].reshape(R_BLK, K, N_CLS)
    v = vals_ref[...]
    rs = rs_ref[...]
    sq = jnp.sum(xg * xg, axis=2)
    g = lax.dot_general(xg, xg, (((2,), (2,)), ((0,), (0,))),
                        preferred_element_type=jnp.float32)
    sqd = sq[:, :, None] + sq[:, None, :] - 2.0 * g
    l2 = jnp.sqrt(jnp.abs(sqd) + EPS)
    dist = jnp.sum(v[:, None, :] * l2, axis=2)
    dist = jnp.where(v == 0.0, FMAX, dist)
    m = jnp.max(-dist, axis=1, keepdims=True)
    e = jnp.exp(-dist - m)
    w = e / jnp.sum(e, axis=1, keepdims=True)
    w = w * v
    w = w / jnp.sum(w, axis=1, keepdims=True)
    out = lax.dot_general(w, xg, (((1,), (1,)), ((0,), (0,))),
                          preferred_element_type=jnp.float32)
    out_ref[...] = rs * out


def _medoid(xg, vals, rs_b, batch_s):
    return pl.pallas_call(
        _medoid_body,
        grid=(batch_s // R_BLK,),
        in_specs=[
            pl.BlockSpec((R_BLK * K, N_CLS), lambda i: (i, 0)),
            pl.BlockSpec((R_BLK, K), lambda i: (i, 0)),
            pl.BlockSpec((R_BLK, N_CLS), lambda i: (i, 0)),
        ],
        out_specs=pl.BlockSpec((R_BLK, N_CLS), lambda i: (i, 0)),
        out_shape=jax.ShapeDtypeStruct((batch_s, N_CLS), jnp.float32),
    )(xg, vals, rs_b)


N_PARTS = 2


def kernel(X, ppr_scores, W0, W1, W2):
    bs = BATCH // N_PARTS
    logits = _mlp(X, W0, W1, W2)
    idx_flat, vals_flat, rs = _sc_topk_kernel(BATCH)(ppr_scores)
    rs_b = jnp.broadcast_to(rs[:, None], (BATCH, N_CLS))
    idx2d = idx_flat.reshape(BATCH * K // CH, CH)
    vals2d = vals_flat.reshape(BATCH, K)
    rows_per_part = bs * K // CH
    outs = []
    for p in range(N_PARTS):
        idx_p = lax.slice_in_dim(idx2d, p * rows_per_part,
                                 (p + 1) * rows_per_part, axis=0)
        xg = _sc_gather_kernel(bs)(logits, idx_p)
        vals_p = lax.slice_in_dim(vals2d, p * bs, (p + 1) * bs, axis=0)
        rs_p = lax.slice_in_dim(rs_b, p * bs, (p + 1) * bs, axis=0)
        outs.append(_medoid(xg, vals_p, rs_p, bs))
    return jnp.concatenate(outs, axis=0)


# speedup vs baseline: 1.2960x; 1.0222x over previous
import functools

import jax
import jax.numpy as jnp
from jax import lax
from jax.experimental import pallas as pl
from jax.experimental.pallas import tpu as pltpu
from jax.experimental.pallas import tpu_sc as plsc

N_NODES = 16384
D_FEAT = 512
HIDDEN = 1024
N_CLS = 128
BATCH = 2048
K = 64
EPS = 100.0 * float(jnp.finfo(jnp.float32).eps)
FMAX = float(jnp.finfo(jnp.float32).max)

NC, NS, L = 2, 16, 16
NW = NC * NS
ROWS_PER_W = BATCH // NW

CAP = 2048
GROUPS_PER_BLK = 16
BUFSZ = CAP + GROUPS_PER_BLK * L + L
N_BLKS = N_NODES // (GROUPS_PER_BLK * L)

T1 = 1.0 - 128.0 / N_NODES
T1_BITS = 0x3F7E0000
CAPF = 256
FGROUPS = (CAPF + 2 * L) // L
FBUF = 544
SENT = N_NODES
RSTRIDE = N_NODES + 128

def _sc_mesh():
    return plsc.VectorSubcoreMesh(
        core_axis_name="c", subcore_axis_name="s", num_cores=NC,
        num_subcores=NS)


def _iota16():
    return lax.iota(jnp.int32, L)


def _splat_i32(x):
    return jnp.full((L,), x, dtype=jnp.int32)


def _select_topk(rowv, base, cand, ptr):
    iota = _iota16()
    ptr_s = jnp.max(ptr)
    ngroups = (ptr_s + (L - 1)) // L

    def count_ge(tvec, strict):
        def body(g, cnt):
            bg = g * L
            idxg = cand[pl.ds(bg, L)]
            valid = (bg + iota) < ptr
            vg = plsc.load_gather(rowv, [idxg + base], mask=valid)
            hit = (vg > tvec if strict else vg >= tvec) & valid
            return cnt + plsc.all_reduce_population_count(hit)
        return lax.fori_loop(0, ngroups, body, _splat_i32(0))

    def bis_body(_, carry):
        lo, hi = carry
        mid = lax.shift_right_arithmetic(lo + hi, 1)
        cnt = count_ge(plsc.bitcast(mid, jnp.float32), False)
        ge = cnt >= K
        return jnp.where(ge, mid, lo), jnp.where(ge, hi, mid)

    lo, _ = lax.fori_loop(
        0, 30, bis_body, (_splat_i32(0), _splat_i32(0x3F800000)))
    tstar = plsc.bitcast(lo, jnp.float32)

    n_gt = count_ge(tstar, True)
    need = K - n_gt

    def compact(g, carry):
        wp, eqc = carry
        bg = g * L
        idxg = cand[pl.ds(bg, L)]
        valid = (bg + iota) < ptr
        vg = plsc.load_gather(rowv, [idxg + base], mask=valid)
        gt = (vg > tstar) & valid
        eq = (vg == tstar) & valid
        eq_rank = eqc + plsc.cumsum(eq.astype(jnp.int32))
        keep = gt | (eq & (eq_rank <= need))
        pos = wp + plsc.cumsum(keep.astype(jnp.int32)) - 1
        plsc.store_scatter(cand, [pos], idxg, mask=keep)
        return (wp + plsc.all_reduce_population_count(keep),
                eqc + plsc.all_reduce_population_count(eq))

    lax.fori_loop(0, ngroups, compact, (_splat_i32(0), _splat_i32(0)))
    return _splat_i32(K), tstar


def _sc_topk_body(rows_pw, ppr_hbm, idx_out, vals_out, rs_out, rowv2, cand,
                  candi, candvv, actg, idx_st, vals_st, rs_st, sem0, sem1):
    wid = lax.axis_index("s") * NC + lax.axis_index("c")
    row0 = wid * rows_pw
    iota = _iota16()
    lane0 = iota == 0
    sems = (sem0, sem1)
    for b in range(2):
        rowv2[pl.ds(b * RSTRIDE + SENT, L)] = jnp.full((L,), -1.0, jnp.float32)
    pltpu.async_copy(ppr_hbm.at[row0], rowv2.at[pl.ds(0, N_NODES)], sem0)

    def row_body(r, base):
        for j in range(FGROUPS):
            candi[pl.ds(j * L, L)] = _splat_i32(SENT)
            actg[pl.ds(j * L, L)] = _splat_i32(SENT // L)

        t1 = jnp.full((L,), T1, jnp.float32)

        def fblk(b, carry):
            acc0, acc1, nact = carry
            cnts = jnp.zeros((L,), jnp.int32)
            for gg in range(GROUPS_PER_BLK):
                off = (b * GROUPS_PER_BLK + gg) * L
                v = rowv2[pl.ds(base + off, L)]
                if gg % 2 == 0:
                    acc0 = acc0 + v
                else:
                    acc1 = acc1 + v
                pc = plsc.all_reduce_population_count(v > t1)
                cnts = jnp.where(iota == gg, pc, cnts)
            act = cnts > 0
            posn = nact + plsc.cumsum(act.astype(jnp.int32)) - 1
            posn = jnp.minimum(posn, FGROUPS * L - 1)
            plsc.store_scatter(actg, [posn], iota + b * L, mask=act)
            return acc0, acc1, nact + plsc.all_reduce_population_count(act)

        acc0, acc1, nact = lax.fori_loop(
            0, N_BLKS, fblk,
            (jnp.zeros((L,), jnp.float32), jnp.zeros((L,), jnp.float32),
             _splat_i32(0)))
        acc = acc0 + acc1
        nact_s = jnp.minimum(jnp.max(nact), FGROUPS * L)

        def cblk(t, ptr):
            av = actg[pl.ds(t * L, L)]
            for j in range(L):
                gb = jnp.clip(av[j], 0, SENT // L) * L
                v = rowv2[pl.ds(base + gb, L)]
                m = v > t1
                pos = ptr + plsc.cumsum(m.astype(jnp.int32)) - 1
                pos = jnp.minimum(pos, FBUF - 1)
                plsc.store_scatter(candi, [pos], iota + gb, mask=m)
                ptr = ptr + plsc.all_reduce_population_count(m)
            return ptr

        ptr = lax.fori_loop(0, (nact_s + L - 1) // L, cblk, _splat_i32(0))
        ptr_s = jnp.max(ptr)
        ok = (ptr_s >= K) & (ptr_s <= CAPF) & (nact_s <= CAPF)

        @pl.when(ok)
        def _fast_select():
            for j in range(FGROUPS):
                ij = candi[pl.ds(j * L, L)]
                candvv[pl.ds(j * L, L)] = plsc.load_gather(rowv2, [ij + base])

            def count_s(tvec, strict):
                cnt = jnp.zeros((L,), jnp.int32)
                for j in range(FGROUPS):
                    vg = candvv[pl.ds(j * L, L)]
                    hit = vg > tvec if strict else vg >= tvec
                    cnt = cnt + hit.astype(jnp.int32)
                return jnp.sum(cnt, axis=0)

            def fbis(_, lohi):
                lo, hi = lohi
                mid = lax.shift_right_arithmetic(lo + hi, 1)
                tmid = lax.bitcast_convert_type(mid, jnp.float32)
                ge = count_s(jnp.full((L,), tmid), False) >= K
                return (jnp.where(ge, mid, lo), jnp.where(ge, hi, mid))

            lo, _ = lax.fori_loop(0, 17, fbis,
                                  (jnp.int32(T1_BITS), jnp.int32(0x3F800000)))
            tstar = jnp.full((L,), lax.bitcast_convert_type(lo, jnp.float32))
            need = K - _splat_i32(count_s(tstar, True))
            ngroups = (ptr_s + (L - 1)) // L

            def comp(g, carry):
                wp, eqc = carry
                base = g * L
                idxg = candi[pl.ds(base, L)]
                vg = candvv[pl.ds(base, L)]
                gt = vg > tstar
                eq = vg == tstar
                eq_rank = eqc + plsc.cumsum(eq.astype(jnp.int32))
                keep = gt | (eq & (eq_rank <= need))
                pos = wp + plsc.cumsum(keep.astype(jnp.int32)) - 1
                plsc.store_scatter(cand, [pos], idxg, mask=keep)
                return (wp + plsc.all_reduce_population_count(keep),
                        eqc + plsc.all_reduce_population_count(eq))

            lax.fori_loop(0, ngroups, comp,
                          (_splat_i32(0), _splat_i32(0)))

        @pl.when(jnp.logical_not(ok))
        def _fallback():
            def blk_body(b, carry):
                fptr, thr = carry
                for gg in range(GROUPS_PER_BLK):
                    off = (b * GROUPS_PER_BLK + gg) * L
                    v = rowv2[pl.ds(base + off, L)]
                    mask = v > thr
                    pos = fptr + plsc.cumsum(mask.astype(jnp.int32)) - 1
                    plsc.store_scatter(cand, [pos], iota + off, mask=mask)
                    fptr = fptr + plsc.all_reduce_population_count(mask)

                def rebuild():
                    return _select_topk(rowv2, base, cand, fptr)

                fptr, thr = lax.cond(jnp.max(fptr) > CAP, rebuild,
                                     lambda: (fptr, thr))
                return fptr, thr

            fptr, _ = lax.fori_loop(
                0, N_BLKS, blk_body,
                (_splat_i32(0), jnp.full((L,), -1.0, jnp.float32)))
            _select_topk(rowv2, base, cand, fptr)

        for j in range(K // L):
            idxg = cand[pl.ds(j * L, L)]
            vg = plsc.load_gather(rowv2, [idxg + base])
            idx_st[pl.ds(r * K + j * L, L)] = idxg
            vals_st[pl.ds(r * K + j * L, L)] = vg

        rsum = jnp.sum(acc, axis=0)
        plsc.store_scatter(rs_st, [_splat_i32(r)],
                           jnp.full((L,), rsum, jnp.float32), mask=lane0)


    def pair(t, _):
        for b in range(2):
            r = t * 2 + b

            @pl.when(r + 1 < rows_pw)
            def _():
                pltpu.async_copy(
                    ppr_hbm.at[row0 + r + 1],
                    rowv2.at[pl.ds(((b + 1) % 2) * RSTRIDE, N_NODES)],
                    sems[(b + 1) % 2])

            pltpu.make_async_copy(ppr_hbm.at[row0 + r],
                                  rowv2.at[pl.ds(b * RSTRIDE, N_NODES)],
                                  sems[b]).wait()
            row_body(r, b * RSTRIDE)
        return 0

    lax.fori_loop(0, rows_pw // 2, pair, 0)
    pltpu.sync_copy(idx_st, idx_out.at[pl.ds(row0 * K, rows_pw * K)])
    pltpu.sync_copy(vals_st, vals_out.at[pl.ds(row0 * K, rows_pw * K)])
    pltpu.sync_copy(rs_st, rs_out.at[pl.ds(row0, rows_pw)])


CH = 128
CHUNKS_PER_W = (BATCH * K) // NW // CH


def _sc_gather_body(chunks_pw, logits_hbm, idx_hbm, out_hbm, idx_v, rows_v,
                    sem0, sem1):
    wid = lax.axis_index("s") * NC + lax.axis_index("c")
    base = wid * chunks_pw * CH
    pltpu.sync_copy(idx_hbm.at[pl.ds(wid * chunks_pw, chunks_pw), :],
                    idx_v)
    sems = (sem0, sem1)
    pltpu.async_copy(logits_hbm.at[idx_v.at[0]], rows_v.at[0], sem0)

    def outer(t, _):
        for b in range(2):
            j = t * 2 + b

            @pl.when(j + 1 < chunks_pw)
            def _():
                pltpu.async_copy(logits_hbm.at[idx_v.at[j + 1]],
                                 rows_v.at[(b + 1) % 2], sems[(b + 1) % 2])

            pltpu.make_async_copy(logits_hbm.at[idx_v.at[j]], rows_v.at[b],
                                  sems[b]).wait()
            pltpu.sync_copy(rows_v.at[b],
                            out_hbm.at[pl.ds(base + j * CH, CH), :])
        return 0

    lax.fori_loop(0, chunks_pw // 2, outer, 0)


_SC_PARAMS = None


def _sc_params():
    return pltpu.CompilerParams(needs_layout_passes=False)


@functools.cache
def _sc_topk_kernel(batch_s):
    rows_pw = batch_s // NW
    return pl.kernel(
        functools.partial(_sc_topk_body, rows_pw),
        compiler_params=_sc_params(),
        out_type=(jax.ShapeDtypeStruct((batch_s * K,), jnp.int32),
                  jax.ShapeDtypeStruct((batch_s * K,), jnp.float32),
                  jax.ShapeDtypeStruct((batch_s,), jnp.float32)),
        mesh=_sc_mesh(),
        scratch_types=[
            pltpu.VMEM((2 * RSTRIDE,), jnp.float32),
            pltpu.VMEM((BUFSZ,), jnp.int32),
            pltpu.VMEM((FBUF,), jnp.int32),
            pltpu.VMEM((FGROUPS * L,), jnp.float32),
            pltpu.VMEM((FGROUPS * L,), jnp.int32),
            pltpu.VMEM((rows_pw * K,), jnp.int32),
            pltpu.VMEM((rows_pw * K,), jnp.float32),
            pltpu.VMEM((rows_pw,), jnp.float32),
            pltpu.SemaphoreType.DMA,
            pltpu.SemaphoreType.DMA,
        ],
    )


@functools.cache
def _sc_gather_kernel(batch_s):
    chunks_pw = (batch_s * K) // NW // CH
    return pl.kernel(
        functools.partial(_sc_gather_body, chunks_pw),
        compiler_params=_sc_params(),
        out_type=jax.ShapeDtypeStruct((batch_s * K, N_CLS), jnp.float32),
        mesh=_sc_mesh(),
        scratch_types=[
            pltpu.VMEM((chunks_pw, CH), jnp.int32),
            pltpu.VMEM((2, CH, N_CLS), jnp.float32),
            pltpu.SemaphoreType.DMA,
            pltpu.SemaphoreType.DMA,
        ],
    )


MLP_BLK = 512


def _mlp_body(x_ref, w0_ref, w1_ref, w2_ref, out_ref):
    h = jnp.dot(x_ref[...], w0_ref[...], preferred_element_type=jnp.float32)
    h = jnp.dot(jnp.maximum(h, 0.0), w1_ref[...],
                preferred_element_type=jnp.float32)
    out_ref[...] = jnp.dot(jnp.maximum(h, 0.0), w2_ref[...],
                           preferred_element_type=jnp.float32)


def _mlp(X, W0, W1, W2):
    return pl.pallas_call(
        _mlp_body,
        grid=(N_NODES // MLP_BLK,),
        in_specs=[
            pl.BlockSpec((MLP_BLK, D_FEAT), lambda i: (i, 0)),
            pl.BlockSpec((D_FEAT, HIDDEN), lambda i: (0, 0)),
            pl.BlockSpec((HIDDEN, HIDDEN), lambda i: (0, 0)),
            pl.BlockSpec((HIDDEN, N_CLS), lambda i: (0, 0)),
        ],
        out_specs=pl.BlockSpec((MLP_BLK, N_CLS), lambda i: (i, 0)),
        out_shape=jax.ShapeDtypeStruct((N_NODES, N_CLS), jnp.float32),
    )(X, W0, W1, W2)


R_BLK = 128


def _medoid_body(xg_ref, vals_ref, rs_ref, out_ref):
    xg = xg_ref[...].reshape(R_BLK, K, N_CLS)
    v = vals_ref[...]
    rs = rs_ref[...]
    sq = jnp.sum(xg * xg, axis=2)
    g = lax.dot_general(xg, xg, (((2,), (2,)), ((0,), (0,))),
                        preferred_element_type=jnp.float32)
    sqd = sq[:, :, None] + sq[:, None, :] - 2.0 * g
    l2 = jnp.sqrt(jnp.abs(sqd) + EPS)
    dist = jnp.sum(v[:, None, :] * l2, axis=2)
    dist = jnp.where(v == 0.0, FMAX, dist)
    m = jnp.max(-dist, axis=1, keepdims=True)
    e = jnp.exp(-dist - m)
    w = e / jnp.sum(e, axis=1, keepdims=True)
    w = w * v
    w = w / jnp.sum(w, axis=1, keepdims=True)
    out = lax.dot_general(w, xg, (((1,), (1,)), ((0,), (0,))),
                          preferred_element_type=jnp.float32)
    out_ref[...] = rs * out


def _medoid(xg, vals, rs_b, batch_s):
    return pl.pallas_call(
        _medoid_body,
        grid=(batch_s // R_BLK,),
        in_specs=[
            pl.BlockSpec((R_BLK * K, N_CLS), lambda i: (i, 0)),
            pl.BlockSpec((R_BLK, K), lambda i: (i, 0)),
            pl.BlockSpec((R_BLK, N_CLS), lambda i: (i, 0)),
        ],
        out_specs=pl.BlockSpec((R_BLK, N_CLS), lambda i: (i, 0)),
        out_shape=jax.ShapeDtypeStruct((batch_s, N_CLS), jnp.float32),
    )(xg, vals, rs_b)


N_PARTS = 2


def kernel(X, ppr_scores, W0, W1, W2):
    bs = BATCH // N_PARTS
    logits = _mlp(X, W0, W1, W2)
    idx_flat, vals_flat, rs = _sc_topk_kernel(BATCH)(ppr_scores)
    rs_b = jnp.broadcast_to(rs[:, None], (BATCH, N_CLS))
    idx2d = idx_flat.reshape(BATCH * K // CH, CH)
    vals2d = vals_flat.reshape(BATCH, K)
    rows_per_part = bs * K // CH
    outs = []
    for p in range(N_PARTS):
        idx_p = lax.slice_in_dim(idx2d, p * rows_per_part,
                                 (p + 1) * rows_per_part, axis=0)
        xg = _sc_gather_kernel(bs)(logits, idx_p)
        vals_p = lax.slice_in_dim(vals2d, p * bs, (p + 1) * bs, axis=0)
        rs_p = lax.slice_in_dim(rs_b, p * bs, (p + 1) * bs, axis=0)
        outs.append(_medoid(xg, vals_p, rs_p, bs))
    return jnp.concatenate(outs, axis=0)


# speedup vs baseline: 1.3026x; 1.0051x over previous
import functools

import jax
import jax.numpy as jnp
from jax import lax
from jax.experimental import pallas as pl
from jax.experimental.pallas import tpu as pltpu
from jax.experimental.pallas import tpu_sc as plsc

N_NODES = 16384
D_FEAT = 512
HIDDEN = 1024
N_CLS = 128
BATCH = 2048
K = 64
EPS = 100.0 * float(jnp.finfo(jnp.float32).eps)
FMAX = float(jnp.finfo(jnp.float32).max)

NC, NS, L = 2, 16, 16
NW = NC * NS
ROWS_PER_W = BATCH // NW

CAP = 2048
GROUPS_PER_BLK = 16
BUFSZ = CAP + GROUPS_PER_BLK * L + L
N_BLKS = N_NODES // (GROUPS_PER_BLK * L)

T1 = 1.0 - 128.0 / N_NODES
T1_BITS = 0x3F7E0000
CAPF = 256
FGROUPS = (CAPF + 2 * L) // L
FBUF = 544
SENT = N_NODES
RSTRIDE = N_NODES + 128

def _sc_mesh():
    return plsc.VectorSubcoreMesh(
        core_axis_name="c", subcore_axis_name="s", num_cores=NC,
        num_subcores=NS)


def _iota16():
    return lax.iota(jnp.int32, L)


def _splat_i32(x):
    return jnp.full((L,), x, dtype=jnp.int32)


def _select_topk(rowv, base, cand, ptr):
    iota = _iota16()
    ptr_s = jnp.max(ptr)
    ngroups = (ptr_s + (L - 1)) // L

    def count_ge(tvec, strict):
        def body(g, cnt):
            bg = g * L
            idxg = cand[pl.ds(bg, L)]
            valid = (bg + iota) < ptr
            vg = plsc.load_gather(rowv, [idxg + base], mask=valid)
            hit = (vg > tvec if strict else vg >= tvec) & valid
            return cnt + plsc.all_reduce_population_count(hit)
        return lax.fori_loop(0, ngroups, body, _splat_i32(0))

    def bis_body(_, carry):
        lo, hi = carry
        mid = lax.shift_right_arithmetic(lo + hi, 1)
        cnt = count_ge(plsc.bitcast(mid, jnp.float32), False)
        ge = cnt >= K
        return jnp.where(ge, mid, lo), jnp.where(ge, hi, mid)

    lo, _ = lax.fori_loop(
        0, 30, bis_body, (_splat_i32(0), _splat_i32(0x3F800000)))
    tstar = plsc.bitcast(lo, jnp.float32)

    n_gt = count_ge(tstar, True)
    need = K - n_gt

    def compact(g, carry):
        wp, eqc = carry
        bg = g * L
        idxg = cand[pl.ds(bg, L)]
        valid = (bg + iota) < ptr
        vg = plsc.load_gather(rowv, [idxg + base], mask=valid)
        gt = (vg > tstar) & valid
        eq = (vg == tstar) & valid
        eq_rank = eqc + plsc.cumsum(eq.astype(jnp.int32))
        keep = gt | (eq & (eq_rank <= need))
        pos = wp + plsc.cumsum(keep.astype(jnp.int32)) - 1
        plsc.store_scatter(cand, [pos], idxg, mask=keep)
        return (wp + plsc.all_reduce_population_count(keep),
                eqc + plsc.all_reduce_population_count(eq))

    lax.fori_loop(0, ngroups, compact, (_splat_i32(0), _splat_i32(0)))
    return _splat_i32(K), tstar


def _sc_topk_body(rows_pw, ppr_hbm, idx_out, vals_out, rs_out, rowv2, cand,
                  candi, candvv, actg, idx_st, vals_st, rs_st, sem0, sem1):
    wid = lax.axis_index("s") * NC + lax.axis_index("c")
    row0 = wid * rows_pw
    iota = _iota16()
    lane0 = iota == 0
    sems = (sem0, sem1)
    for b in range(2):
        rowv2[pl.ds(b * RSTRIDE + SENT, L)] = jnp.full((L,), -1.0, jnp.float32)
    pltpu.async_copy(ppr_hbm.at[row0], rowv2.at[pl.ds(0, N_NODES)], sem0)

    def row_body(r, base):
        for j in range(FGROUPS):
            candi[pl.ds(j * L, L)] = _splat_i32(SENT)
            actg[pl.ds(j * L, L)] = _splat_i32(SENT // L)

        t1 = jnp.full((L,), T1, jnp.float32)

        def fblk(b, carry):
            acc0, acc1, nact = carry
            cnts = jnp.zeros((L,), jnp.int32)
            for gg in range(GROUPS_PER_BLK):
                off = (b * GROUPS_PER_BLK + gg) * L
                v = rowv2[pl.ds(base + off, L)]
                if gg % 2 == 0:
                    acc0 = acc0 + v
                else:
                    acc1 = acc1 + v
                pc = plsc.all_reduce_population_count(v > t1)
                cnts = jnp.where(iota == gg, pc, cnts)
            act = cnts > 0
            posn = nact + plsc.cumsum(act.astype(jnp.int32)) - 1
            posn = jnp.minimum(posn, FGROUPS * L - 1)
            plsc.store_scatter(actg, [posn], iota + b * L, mask=act)
            return acc0, acc1, nact + plsc.all_reduce_population_count(act)

        acc0, acc1, nact = lax.fori_loop(
            0, N_BLKS, fblk,
            (jnp.zeros((L,), jnp.float32), jnp.zeros((L,), jnp.float32),
             _splat_i32(0)))
        acc = acc0 + acc1
        nact_s = jnp.minimum(jnp.max(nact), FGROUPS * L)

        def cblk(t, ptr):
            av = actg[pl.ds(t * L, L)]
            for j in range(L):
                gb = jnp.clip(av[j], 0, SENT // L) * L
                v = rowv2[pl.ds(base + gb, L)]
                m = v > t1
                pos = ptr + plsc.cumsum(m.astype(jnp.int32)) - 1
                pos = jnp.minimum(pos, FBUF - 1)
                plsc.store_scatter(candi, [pos], iota + gb, mask=m)
                ptr = ptr + plsc.all_reduce_population_count(m)
            return ptr

        ptr = lax.fori_loop(0, (nact_s + L - 1) // L, cblk, _splat_i32(0))
        ptr_s = jnp.max(ptr)
        ok = (ptr_s >= K) & (ptr_s <= CAPF) & (nact_s <= CAPF)

        @pl.when(ok)
        def _fast_select():
            for j in range(FGROUPS):
                ij = candi[pl.ds(j * L, L)]
                candvv[pl.ds(j * L, L)] = plsc.load_gather(rowv2, [ij + base])

            def count_s(tvec, strict):
                cnt = jnp.zeros((L,), jnp.int32)
                for j in range(FGROUPS):
                    vg = candvv[pl.ds(j * L, L)]
                    hit = vg > tvec if strict else vg >= tvec
                    cnt = cnt + hit.astype(jnp.int32)
                return jnp.sum(cnt, axis=0)

            def fbis(_, lohi):
                lo, hi = lohi
                mid = lax.shift_right_arithmetic(lo + hi, 1)
                tmid = lax.bitcast_convert_type(mid, jnp.float32)
                ge = count_s(jnp.full((L,), tmid), False) >= K
                return (jnp.where(ge, mid, lo), jnp.where(ge, hi, mid))

            lo, _ = lax.fori_loop(0, 17, fbis,
                                  (jnp.int32(T1_BITS), jnp.int32(0x3F800000)))
            tstar = jnp.full((L,), lax.bitcast_convert_type(lo, jnp.float32))
            need = K - _splat_i32(count_s(tstar, True))
            ngroups = (ptr_s + (L - 1)) // L

            def comp(g, carry):
                wp, eqc = carry
                base = g * L
                idxg = candi[pl.ds(base, L)]
                vg = candvv[pl.ds(base, L)]
                gt = vg > tstar
                eq = vg == tstar
                eq_rank = eqc + plsc.cumsum(eq.astype(jnp.int32))
                keep = gt | (eq & (eq_rank <= need))
                pos = wp + plsc.cumsum(keep.astype(jnp.int32)) - 1
                plsc.store_scatter(cand, [pos], idxg, mask=keep)
                return (wp + plsc.all_reduce_population_count(keep),
                        eqc + plsc.all_reduce_population_count(eq))

            lax.fori_loop(0, ngroups, comp,
                          (_splat_i32(0), _splat_i32(0)))

        @pl.when(jnp.logical_not(ok))
        def _fallback():
            def blk_body(b, carry):
                fptr, thr = carry
                for gg in range(GROUPS_PER_BLK):
                    off = (b * GROUPS_PER_BLK + gg) * L
                    v = rowv2[pl.ds(base + off, L)]
                    mask = v > thr
                    pos = fptr + plsc.cumsum(mask.astype(jnp.int32)) - 1
                    plsc.store_scatter(cand, [pos], iota + off, mask=mask)
                    fptr = fptr + plsc.all_reduce_population_count(mask)

                def rebuild():
                    return _select_topk(rowv2, base, cand, fptr)

                fptr, thr = lax.cond(jnp.max(fptr) > CAP, rebuild,
                                     lambda: (fptr, thr))
                return fptr, thr

            fptr, _ = lax.fori_loop(
                0, N_BLKS, blk_body,
                (_splat_i32(0), jnp.full((L,), -1.0, jnp.float32)))
            _select_topk(rowv2, base, cand, fptr)

        for j in range(K // L):
            idxg = cand[pl.ds(j * L, L)]
            vg = plsc.load_gather(rowv2, [idxg + base])
            idx_st[pl.ds(r * K + j * L, L)] = idxg
            vals_st[pl.ds(r * K + j * L, L)] = vg

        rsum = jnp.sum(acc, axis=0)
        plsc.store_scatter(rs_st, [_splat_i32(r)],
                           jnp.full((L,), rsum, jnp.float32), mask=lane0)


    def pair(t, _):
        for b in range(2):
            r = t * 2 + b

            @pl.when(r + 1 < rows_pw)
            def _():
                pltpu.async_copy(
                    ppr_hbm.at[row0 + r + 1],
                    rowv2.at[pl.ds(((b + 1) % 2) * RSTRIDE, N_NODES)],
                    sems[(b + 1) % 2])

            pltpu.make_async_copy(ppr_hbm.at[row0 + r],
                                  rowv2.at[pl.ds(b * RSTRIDE, N_NODES)],
                                  sems[b]).wait()
            row_body(r, b * RSTRIDE)
        return 0

    lax.fori_loop(0, rows_pw // 2, pair, 0)
    pltpu.sync_copy(idx_st, idx_out.at[pl.ds(row0 * K, rows_pw * K)])
    pltpu.sync_copy(vals_st, vals_out.at[pl.ds(row0 * K, rows_pw * K)])
    pltpu.sync_copy(rs_st, rs_out.at[pl.ds(row0, rows_pw)])


CH = 128
CHUNKS_PER_W = (BATCH * K) // NW // CH


def _sc_gather_body(chunks_pw, logits_hbm, idx_hbm, out_hbm, idx_v, rows_v,
                    sem0, sem1):
    wid = lax.axis_index("s") * NC + lax.axis_index("c")
    base = wid * chunks_pw * CH
    pltpu.sync_copy(idx_hbm.at[pl.ds(wid * chunks_pw, chunks_pw), :],
                    idx_v)
    sems = (sem0, sem1)
    pltpu.async_copy(logits_hbm.at[idx_v.at[0]], rows_v.at[0], sem0)

    def outer(t, _):
        for b in range(2):
            j = t * 2 + b

            @pl.when(j + 1 < chunks_pw)
            def _():
                pltpu.async_copy(logits_hbm.at[idx_v.at[j + 1]],
                                 rows_v.at[(b + 1) % 2], sems[(b + 1) % 2])

            pltpu.make_async_copy(logits_hbm.at[idx_v.at[j]], rows_v.at[b],
                                  sems[b]).wait()
            pltpu.sync_copy(rows_v.at[b],
                            out_hbm.at[pl.ds(base + j * CH, CH), :])
        return 0

    lax.fori_loop(0, chunks_pw // 2, outer, 0)


_SC_PARAMS = None


def _sc_params():
    return pltpu.CompilerParams(needs_layout_passes=False)


@functools.cache
def _sc_topk_kernel(batch_s):
    rows_pw = batch_s // NW
    return pl.kernel(
        functools.partial(_sc_topk_body, rows_pw),
        compiler_params=_sc_params(),
        out_type=(jax.ShapeDtypeStruct((batch_s * K,), jnp.int32),
                  jax.ShapeDtypeStruct((batch_s * K,), jnp.float32),
                  jax.ShapeDtypeStruct((batch_s,), jnp.float32)),
        mesh=_sc_mesh(),
        scratch_types=[
            pltpu.VMEM((2 * RSTRIDE,), jnp.float32),
            pltpu.VMEM((BUFSZ,), jnp.int32),
            pltpu.VMEM((FBUF,), jnp.int32),
            pltpu.VMEM((FGROUPS * L,), jnp.float32),
            pltpu.VMEM((FGROUPS * L,), jnp.int32),
            pltpu.VMEM((rows_pw * K,), jnp.int32),
            pltpu.VMEM((rows_pw * K,), jnp.float32),
            pltpu.VMEM((rows_pw,), jnp.float32),
            pltpu.SemaphoreType.DMA,
            pltpu.SemaphoreType.DMA,
        ],
    )


@functools.cache
def _sc_gather_kernel(batch_s):
    chunks_pw = (batch_s * K) // NW // CH
    return pl.kernel(
        functools.partial(_sc_gather_body, chunks_pw),
        compiler_params=_sc_params(),
        out_type=jax.ShapeDtypeStruct((batch_s * K, N_CLS), jnp.float32),
        mesh=_sc_mesh(),
        scratch_types=[
            pltpu.VMEM((chunks_pw, CH), jnp.int32),
            pltpu.VMEM((2, CH, N_CLS), jnp.float32),
            pltpu.SemaphoreType.DMA,
            pltpu.SemaphoreType.DMA,
        ],
    )


MLP_BLK = 512


def _mlp_body(x_ref, w0_ref, w1_ref, w2_ref, out_ref):
    h = jnp.dot(x_ref[...], w0_ref[...], preferred_element_type=jnp.float32)
    h = jnp.dot(jnp.maximum(h, 0.0), w1_ref[...],
                preferred_element_type=jnp.float32)
    out_ref[...] = jnp.dot(jnp.maximum(h, 0.0), w2_ref[...],
                           preferred_element_type=jnp.float32)


def _mlp(X, W0, W1, W2):
    return pl.pallas_call(
        _mlp_body,
        grid=(N_NODES // MLP_BLK,),
        in_specs=[
            pl.BlockSpec((MLP_BLK, D_FEAT), lambda i: (i, 0)),
            pl.BlockSpec((D_FEAT, HIDDEN), lambda i: (0, 0)),
            pl.BlockSpec((HIDDEN, HIDDEN), lambda i: (0, 0)),
            pl.BlockSpec((HIDDEN, N_CLS), lambda i: (0, 0)),
        ],
        out_specs=pl.BlockSpec((MLP_BLK, N_CLS), lambda i: (i, 0)),
        out_shape=jax.ShapeDtypeStruct((N_NODES, N_CLS), jnp.float32),
    )(X, W0, W1, W2)


R_BLK = 256


def _medoid_body(xg_ref, vals_ref, rs_ref, out_ref):
    xg = xg_ref[...].reshape(R_BLK, K, N_CLS)
    v = vals_ref[...]
    rs = rs_ref[...]
    sq = jnp.sum(xg * xg, axis=2)
    g = lax.dot_general(xg, xg, (((2,), (2,)), ((0,), (0,))),
                        preferred_element_type=jnp.float32)
    sqd = sq[:, :, None] + sq[:, None, :] - 2.0 * g
    l2 = jnp.sqrt(jnp.abs(sqd) + EPS)
    dist = jnp.sum(v[:, None, :] * l2, axis=2)
    dist = jnp.where(v == 0.0, FMAX, dist)
    m = jnp.max(-dist, axis=1, keepdims=True)
    e = jnp.exp(-dist - m)
    w = e / jnp.sum(e, axis=1, keepdims=True)
    w = w * v
    w = w / jnp.sum(w, axis=1, keepdims=True)
    out = lax.dot_general(w, xg, (((1,), (1,)), ((0,), (0,))),
                          preferred_element_type=jnp.float32)
    out_ref[...] = rs * out


def _medoid(xg, vals, rs_b, batch_s):
    return pl.pallas_call(
        _medoid_body,
        grid=(batch_s // R_BLK,),
        in_specs=[
            pl.BlockSpec((R_BLK * K, N_CLS), lambda i: (i, 0)),
            pl.BlockSpec((R_BLK, K), lambda i: (i, 0)),
            pl.BlockSpec((R_BLK, N_CLS), lambda i: (i, 0)),
        ],
        out_specs=pl.BlockSpec((R_BLK, N_CLS), lambda i: (i, 0)),
        out_shape=jax.ShapeDtypeStruct((batch_s, N_CLS), jnp.float32),
    )(xg, vals, rs_b)


N_PARTS = 2


def kernel(X, ppr_scores, W0, W1, W2):
    bs = BATCH // N_PARTS
    logits = _mlp(X, W0, W1, W2)
    idx_flat, vals_flat, rs = _sc_topk_kernel(BATCH)(ppr_scores)
    rs_b = jnp.broadcast_to(rs[:, None], (BATCH, N_CLS))
    idx2d = idx_flat.reshape(BATCH * K // CH, CH)
    vals2d = vals_flat.reshape(BATCH, K)
    rows_per_part = bs * K // CH
    outs = []
    for p in range(N_PARTS):
        idx_p = lax.slice_in_dim(idx2d, p * rows_per_part,
                                 (p + 1) * rows_per_part, axis=0)
        xg = _sc_gather_kernel(bs)(logits, idx_p)
        vals_p = lax.slice_in_dim(vals2d, p * bs, (p + 1) * bs, axis=0)
        rs_p = lax.slice_in_dim(rs_b, p * bs, (p + 1) * bs, axis=0)
        outs.append(_medoid(xg, vals_p, rs_p, bs))
    return jnp.concatenate(outs, axis=0)
